# Initial kernel scaffold; baseline (speedup 1.0000x reference)
import jax
import jax.numpy as jnp
from jax.experimental import pallas as pl


def kernel(x, edge_index, edge_attr, emb_table, W_in, b_in, W_lin, b_lin, W1, b1, W2, b2, W3, b3):
    raise NotImplementedError("write your pallas kernel here")


# speedup vs baseline: 6.2608x; 6.2608x over previous
import functools

import jax
import jax.numpy as jnp
from jax import lax
from jax.experimental import pallas as pl
from jax.experimental.pallas import tpu as pltpu
from jax.experimental.pallas import tpu_sc as plsc

N = 50000
E = 800000
EMB = 64
HID = 128

NC = 2
NS = 16
TILES = NC * NS

N_PAD = 53248
E_PAD = 819200

NPT32 = N_PAD // TILES
NPT16 = N_PAD // NS
EPT16 = E_PAD // NS
EPT32 = E_PAD // TILES

_MESH = plsc.VectorSubcoreMesh(core_axis_name="c", subcore_axis_name="s")


def _rsqrt16(v):
    i = lax.bitcast_convert_type(v, jnp.int32)
    i = jnp.int32(0x5F3759DF) - (i >> 1)
    y = lax.bitcast_convert_type(i, jnp.float32)
    for _ in range(3):
        y = y * (jnp.float32(1.5) - jnp.float32(0.5) * v * y * y)
    return jnp.where(v > jnp.float32(0.0), y, jnp.float32(0.0))


@functools.partial(
    pl.kernel,
    out_type=(
        jax.ShapeDtypeStruct((N_PAD,), jnp.float32),
        jax.ShapeDtypeStruct((2 * N_PAD, 32), jnp.float32),
    ),
    mesh=_MESH,
    compiler_params=pltpu.CompilerParams(use_tc_tiling_on_sc=False),
    scratch_types=[
        pltpu.VMEM_SHARED((N_PAD,), jnp.float32),
        pltpu.VMEM((10240,), jnp.float32),
        pltpu.VMEM((80, 128), jnp.int32),
        pltpu.VMEM((NPT32,), jnp.float32),
        pltpu.VMEM((NPT32,), jnp.float32),
        pltpu.VMEM((NPT32,), jnp.int32),
        pltpu.VMEM((128, 64), jnp.float32),
        pltpu.VMEM((128, 32), jnp.float32),
        pltpu.VMEM((128, 32), jnp.float32),
        pltpu.SemaphoreType.DMA,
    ],
)
def _sc_prep(emb_hbm, x_hbm, dst2d_hbm, ea_hbm, zn_hbm,
             dis_hbm, embcat_hbm,
             deg_acc, eabuf, dstbuf, degbuf, disbuf, xbuf, rows, lobuf, hibuf,
             sem):
    c = lax.axis_index("c")
    s = lax.axis_index("s")
    wid = s * NC + c

    pltpu.sync_copy(zn_hbm.at[pl.ds(s * NPT16, NPT16)],
                    deg_acc.at[pl.ds(s * NPT16, NPT16)])
    plsc.subcore_barrier()

    def sg_body(sg, _):
        base = pl.multiple_of(s * EPT16 + sg * 10240, 1024)
        pltpu.sync_copy(ea_hbm.at[pl.ds(base, 10240)], eabuf)
        pltpu.sync_copy(dst2d_hbm.at[pl.ds(pl.multiple_of(base // 128, 8), 80)], dstbuf)

        def g_body(g, _):
            pltpu.sync_copy(eabuf.at[pl.ds(g * 128, 128)],
                            deg_acc.at[dstbuf.at[g]], add=True)
            return 0
        lax.fori_loop(0, 80, g_body, 0)
        return 0
    lax.fori_loop(0, 5, sg_body, 0)
    plsc.subcore_barrier()

    nbase = pl.multiple_of(wid * NPT32, 128)
    pltpu.sync_copy(deg_acc.at[pl.ds(nbase, NPT32)], degbuf)
    pltpu.sync_copy(x_hbm.at[pl.ds(nbase, NPT32)], xbuf)

    def dis_body(i, _):
        disbuf[pl.ds(i * 16, 16)] = _rsqrt16(degbuf[pl.ds(i * 16, 16)])
        return 0
    lax.fori_loop(0, NPT32 // 16, dis_body, 0)
    pltpu.sync_copy(disbuf, dis_hbm.at[pl.ds(nbase, NPT32)])

    def j_body(j, _):
        pltpu.async_copy(emb_hbm.at[xbuf.at[pl.ds(j * 128, 128)]],
                         rows, sem).wait()

        def e_body(e16, _):
            w16 = disbuf[pl.ds(j * 128 + e16 * 16, 16)]
            for u in range(16):
                e = e16 * 16 + u
                w = w16[u]
                lobuf[e, pl.ds(0, 16)] = rows[e, pl.ds(0, 16)] * w
                lobuf[e, pl.ds(16, 16)] = rows[e, pl.ds(16, 16)] * w
                hibuf[e, pl.ds(0, 16)] = rows[e, pl.ds(32, 16)] * w
                hibuf[e, pl.ds(16, 16)] = rows[e, pl.ds(48, 16)] * w
            return 0
        lax.fori_loop(0, 8, e_body, 0)
        pltpu.sync_copy(lobuf, embcat_hbm.at[pl.ds(nbase + j * 128, 128)])
        pltpu.sync_copy(hibuf, embcat_hbm.at[pl.ds(N_PAD + nbase + j * 128, 128)])
        return 0
    lax.fori_loop(0, 13, j_body, 0)


@functools.partial(
    pl.kernel,
    out_type=jax.ShapeDtypeStruct((2 * N_PAD, 32), jnp.float32),
    mesh=_MESH,
    compiler_params=pltpu.CompilerParams(use_tc_tiling_on_sc=False),
    scratch_types=[
        pltpu.VMEM_SHARED((N_PAD, 32), jnp.float32),
        pltpu.VMEM((2048,), jnp.int32),
        pltpu.VMEM((16, 128), jnp.int32),
        pltpu.VMEM((2048,), jnp.float32),
        pltpu.VMEM((128, 32), jnp.float32),
        pltpu.SemaphoreType.DMA,
    ],
)
def _sc_agg(embcat_hbm, src2f_hbm, dst2d_hbm, ea_hbm, znd_hbm,
            aggcat_hbm,
            acc, srcbuf, dstbuf, eabuf, rows, sem):
    c = lax.axis_index("c")
    s = lax.axis_index("s")

    pltpu.sync_copy(znd_hbm.at[pl.ds(s * NPT16, NPT16)],
                    acc.at[pl.ds(s * NPT16, NPT16)])
    plsc.subcore_barrier()

    def sg_body(sg, _):
        base = pl.multiple_of(s * EPT16 + sg * 2048, 1024)
        pltpu.sync_copy(src2f_hbm.at[pl.ds(c * E_PAD + base, 2048)], srcbuf)
        pltpu.sync_copy(ea_hbm.at[pl.ds(base, 2048)], eabuf)
        pltpu.sync_copy(dst2d_hbm.at[pl.ds(pl.multiple_of(base // 128, 8), 16)], dstbuf)

        def g_body(g, _):
            pltpu.async_copy(embcat_hbm.at[srcbuf.at[pl.ds(g * 128, 128)]],
                             rows, sem).wait()

            def e_body(e16, _):
                w16 = eabuf[pl.ds(g * 128 + e16 * 16, 16)]
                for u in range(16):
                    e = e16 * 16 + u
                    w = w16[u]
                    rows[e, pl.ds(0, 16)] = rows[e, pl.ds(0, 16)] * w
                    rows[e, pl.ds(16, 16)] = rows[e, pl.ds(16, 16)] * w
                return 0
            lax.fori_loop(0, 8, e_body, 0)
            pltpu.sync_copy(rows, acc.at[dstbuf.at[g]], add=True)
            return 0
        lax.fori_loop(0, 16, g_body, 0)
        return 0
    lax.fori_loop(0, 25, sg_body, 0)
    plsc.subcore_barrier()

    pltpu.sync_copy(acc.at[pl.ds(s * NPT16, NPT16)],
                    aggcat_hbm.at[pl.ds(c * N_PAD + s * NPT16, NPT16)])


@functools.partial(
    pl.kernel,
    out_type=(
        jax.ShapeDtypeStruct((E_PAD, 64), jnp.float32),
        jax.ShapeDtypeStruct((E_PAD, 64), jnp.float32),
    ),
    mesh=_MESH,
    compiler_params=pltpu.CompilerParams(use_tc_tiling_on_sc=False),
    scratch_types=[
        pltpu.VMEM((512,), jnp.int32),
        pltpu.VMEM((512,), jnp.int32),
        pltpu.VMEM((512, 64), jnp.float32),
        pltpu.VMEM((512, 64), jnp.float32),
        pltpu.SemaphoreType.DMA,
        pltpu.SemaphoreType.DMA,
    ],
)
def _sc_decode(ab_hbm, srca_hbm, dstb_hbm,
               ga_hbm, gb_hbm,
               iabuf, ibbuf, bufa, bufb, sema, semb):
    c = lax.axis_index("c")
    s = lax.axis_index("s")
    wid = s * NC + c
    ebase = pl.multiple_of(wid * EPT32, 512)

    def it_body(it, _):
        pltpu.sync_copy(srca_hbm.at[pl.ds(ebase + it * 512, 512)], iabuf)
        pltpu.sync_copy(dstb_hbm.at[pl.ds(ebase + it * 512, 512)], ibbuf)
        cps = []
        for j in range(4):
            cps.append(pltpu.async_copy(
                ab_hbm.at[iabuf.at[pl.ds(j * 128, 128)]],
                bufa.at[pl.ds(j * 128, 128)], sema))
            cps.append(pltpu.async_copy(
                ab_hbm.at[ibbuf.at[pl.ds(j * 128, 128)]],
                bufb.at[pl.ds(j * 128, 128)], semb))
        for cp in cps:
            cp.wait()
        pltpu.sync_copy(bufa, ga_hbm.at[pl.ds(ebase + it * 512, 512)])
        pltpu.sync_copy(bufb, gb_hbm.at[pl.ds(ebase + it * 512, 512)])
        return 0
    lax.fori_loop(0, 50, it_body, 0)


def _elu(v):
    return jnp.where(v > 0, v, jnp.exp(jnp.minimum(v, 0.0)) - 1.0)


def _tc_dense_body(agglo_ref, agghi_ref, dis_ref, win_ref, bin_ref,
                   wlin_ref, blin_ref, w1_ref, b1_ref, a_ref, b_ref):
    dis = dis_ref[...]
    agg = jnp.concatenate([agglo_ref[...], agghi_ref[...]], axis=1) * dis
    t = _elu(jnp.dot(agg, win_ref[...],
                     preferred_element_type=jnp.float32) + bin_ref[...])
    nodes = _elu(jnp.dot(t, wlin_ref[...],
                         preferred_element_type=jnp.float32) + blin_ref[...])
    w1 = w1_ref[...]
    a_ref[...] = jnp.dot(nodes, w1[0:64, :],
                         preferred_element_type=jnp.float32) + b1_ref[...]
    b_ref[...] = jnp.dot(nodes, w1[64:128, :],
                         preferred_element_type=jnp.float32)


_BN = 2048


def _tc_dense(agg_cat, dis2, w_in, b_in2, w_lin, b_lin2, w1, b12):
    grid = (N_PAD // _BN,)
    full = lambda shape: pl.BlockSpec(shape, lambda i: (0, 0))
    return pl.pallas_call(
        _tc_dense_body,
        grid=grid,
        in_specs=[
            pl.BlockSpec((_BN, 32), lambda i: (i, 0)),
            pl.BlockSpec((_BN, 32), lambda i: (i + N_PAD // _BN, 0)),
            pl.BlockSpec((_BN, 1), lambda i: (i, 0)),
            full((EMB, HID)),
            full((1, HID)),
            full((HID, EMB)),
            full((1, EMB)),
            full((2 * EMB, EMB)),
            full((1, EMB)),
        ],
        out_specs=(
            pl.BlockSpec((_BN, 64), lambda i: (i, 0)),
            pl.BlockSpec((_BN, 64), lambda i: (i, 0)),
        ),
        out_shape=(
            jax.ShapeDtypeStruct((N_PAD, 64), jnp.float32),
            jax.ShapeDtypeStruct((N_PAD, 64), jnp.float32),
        ),
    )(agg_cat, agg_cat, dis2, w_in, b_in2, w_lin, b_lin2, w1, b12)


_BE = 4096


def _tc_mlp_body(ga_ref, gb_ref, w2_ref, b2_ref, w3_ref, b3_ref, out_ref):
    u = jnp.maximum(ga_ref[...] + gb_ref[...], 0.0)
    h = jnp.maximum(jnp.dot(u, w2_ref[...],
                            preferred_element_type=jnp.float32) + b2_ref[...],
                    0.0)
    out_ref[...] = jnp.sum(h * w3_ref[...], axis=1) + b3_ref[0, 0]


def _tc_mlp(ga, gb, w2, b22, w3r, b32):
    grid = (E_PAD // _BE,)
    full = lambda shape: pl.BlockSpec(shape, lambda i: (0, 0))
    return pl.pallas_call(
        _tc_mlp_body,
        grid=grid,
        in_specs=[
            pl.BlockSpec((_BE, 64), lambda i: (i, 0)),
            pl.BlockSpec((_BE, 64), lambda i: (i, 0)),
            full((EMB, EMB)),
            full((1, EMB)),
            full((1, EMB)),
            full((1, 1)),
        ],
        out_specs=pl.BlockSpec((_BE,), lambda i: (i,)),
        out_shape=jax.ShapeDtypeStruct((E_PAD,), jnp.float32),
    )(ga, gb, w2, b22, w3r, b32)


def kernel(x, edge_index, edge_attr, emb_table, W_in, b_in, W_lin, b_lin,
           W1, b1, W2, b2, W3, b3):
    src = edge_index[0]
    dst = edge_index[1]

    epad = E_PAD - E
    src_p = jnp.concatenate([src, jnp.zeros((epad,), jnp.int32)])
    dst_p = jnp.concatenate([dst, jnp.zeros((epad,), jnp.int32)])
    ea_p = jnp.concatenate([edge_attr, jnp.zeros((epad,), jnp.float32)])
    x_p = jnp.concatenate([x, jnp.zeros((N_PAD - N,), jnp.int32)])

    dst2d = dst_p.reshape(E_PAD // 128, 128)
    src2f = jnp.concatenate([src_p, src_p + N_PAD])
    dstb = dst_p + N_PAD

    zn = jnp.zeros((N_PAD,), jnp.float32)
    znd = jnp.zeros((N_PAD, 32), jnp.float32)

    dis, emb_cat = _sc_prep(emb_table, x_p, dst2d, ea_p, zn)
    agg_cat = _sc_agg(emb_cat, src2f, dst2d, ea_p, znd)

    a_mat, b_mat = _tc_dense(
        agg_cat, dis.reshape(N_PAD, 1),
        W_in, b_in.reshape(1, HID),
        W_lin, b_lin.reshape(1, EMB),
        W1, b1.reshape(1, EMB))

    ab = jnp.concatenate([a_mat, b_mat], axis=0)
    ga, gb = _sc_decode(ab, src_p, dstb)

    out = _tc_mlp(ga, gb, W2, b2.reshape(1, EMB),
                  W3.reshape(1, EMB), b3.reshape(1, 1))
    return out[:E]


# speedup vs baseline: 7.5036x; 1.1985x over previous
import functools

import jax
import jax.numpy as jnp
from jax import lax
from jax.experimental import pallas as pl
from jax.experimental.pallas import tpu as pltpu
from jax.experimental.pallas import tpu_sc as plsc

N = 50000
E = 800000
EMB = 64
HID = 128

NC = 2
NS = 16
TILES = NC * NS

N_PAD = 53248
E_PAD = 819200

NPT32 = N_PAD // TILES
NPT16 = N_PAD // NS
EPT16 = E_PAD // NS
EPT32 = E_PAD // TILES

_MESH = plsc.VectorSubcoreMesh(core_axis_name="c", subcore_axis_name="s")


def _rsqrt16(v):
    i = lax.bitcast_convert_type(v, jnp.int32)
    i = jnp.int32(0x5F3759DF) - (i >> 1)
    y = lax.bitcast_convert_type(i, jnp.float32)
    for _ in range(3):
        y = y * (jnp.float32(1.5) - jnp.float32(0.5) * v * y * y)
    return jnp.where(v > jnp.float32(0.0), y, jnp.float32(0.0))


@functools.partial(
    pl.kernel,
    out_type=(
        jax.ShapeDtypeStruct((N_PAD,), jnp.float32),
        jax.ShapeDtypeStruct((2 * N_PAD, 32), jnp.float32),
    ),
    mesh=_MESH,
    compiler_params=pltpu.CompilerParams(use_tc_tiling_on_sc=False),
    scratch_types=[
        pltpu.VMEM_SHARED((N_PAD,), jnp.float32),
        pltpu.VMEM((10240,), jnp.float32),
        pltpu.VMEM((80, 128), jnp.int32),
        pltpu.VMEM((NPT32,), jnp.float32),
        pltpu.VMEM((NPT32,), jnp.float32),
        pltpu.VMEM((NPT32,), jnp.int32),
        pltpu.VMEM((128, 64), jnp.float32),
        pltpu.VMEM((128, 32), jnp.float32),
        pltpu.VMEM((128, 32), jnp.float32),
        pltpu.SemaphoreType.DMA,
    ],
)
def _sc_prep(emb_hbm, x_hbm, dst2d_hbm, ea_hbm, zn_hbm,
             dis_hbm, embcat_hbm,
             deg_acc, eabuf, dstbuf, degbuf, disbuf, xbuf, rows, lobuf, hibuf,
             sem):
    c = lax.axis_index("c")
    s = lax.axis_index("s")
    wid = s * NC + c

    pltpu.sync_copy(zn_hbm.at[pl.ds(s * NPT16, NPT16)],
                    deg_acc.at[pl.ds(s * NPT16, NPT16)])
    plsc.subcore_barrier()

    def sg_body(sg, _):
        base = pl.multiple_of(s * EPT16 + sg * 10240, 1024)
        pltpu.sync_copy(ea_hbm.at[pl.ds(base, 10240)], eabuf)
        pltpu.sync_copy(dst2d_hbm.at[pl.ds(pl.multiple_of(base // 128, 8), 80)], dstbuf)

        def g_fire(g, _):
            pltpu.async_copy(eabuf.at[pl.ds(g * 128, 128)],
                             deg_acc.at[dstbuf.at[g]], sem, add=True)
            return 0
        lax.fori_loop(0, 80, g_fire, 0)

        def g_drain(g, _):
            pltpu.make_async_copy(eabuf.at[pl.ds(0, 128)],
                                  deg_acc.at[dstbuf.at[0]], sem).wait()
            return 0
        lax.fori_loop(0, 80, g_drain, 0)
        return 0
    lax.fori_loop(0, 5, sg_body, 0)
    plsc.subcore_barrier()

    nbase = pl.multiple_of(wid * NPT32, 128)
    pltpu.sync_copy(deg_acc.at[pl.ds(nbase, NPT32)], degbuf)
    pltpu.sync_copy(x_hbm.at[pl.ds(nbase, NPT32)], xbuf)

    def dis_body(i, _):
        disbuf[pl.ds(i * 16, 16)] = _rsqrt16(degbuf[pl.ds(i * 16, 16)])
        return 0
    lax.fori_loop(0, NPT32 // 16, dis_body, 0)
    pltpu.sync_copy(disbuf, dis_hbm.at[pl.ds(nbase, NPT32)])

    def j_body(j, _):
        pltpu.async_copy(emb_hbm.at[xbuf.at[pl.ds(j * 128, 128)]],
                         rows, sem).wait()

        def e_body(e16, _):
            w16 = disbuf[pl.ds(j * 128 + e16 * 16, 16)]
            for u in range(16):
                e = e16 * 16 + u
                w = w16[u]
                lobuf[e, pl.ds(0, 16)] = rows[e, pl.ds(0, 16)] * w
                lobuf[e, pl.ds(16, 16)] = rows[e, pl.ds(16, 16)] * w
                hibuf[e, pl.ds(0, 16)] = rows[e, pl.ds(32, 16)] * w
                hibuf[e, pl.ds(16, 16)] = rows[e, pl.ds(48, 16)] * w
            return 0
        lax.fori_loop(0, 8, e_body, 0)
        pltpu.sync_copy(lobuf, embcat_hbm.at[pl.ds(nbase + j * 128, 128)])
        pltpu.sync_copy(hibuf, embcat_hbm.at[pl.ds(N_PAD + nbase + j * 128, 128)])
        return 0
    lax.fori_loop(0, 13, j_body, 0)


@functools.partial(
    pl.kernel,
    out_type=jax.ShapeDtypeStruct((2 * N_PAD, 32), jnp.float32),
    mesh=_MESH,
    compiler_params=pltpu.CompilerParams(use_tc_tiling_on_sc=False),
    scratch_types=[
        pltpu.VMEM_SHARED((N_PAD, 32), jnp.float32),
        pltpu.VMEM((2048,), jnp.int32),
        pltpu.VMEM((16, 128), jnp.int32),
        pltpu.VMEM((2048,), jnp.float32),
        pltpu.VMEM((128, 32), jnp.float32),
        pltpu.VMEM((128, 32), jnp.float32),
        pltpu.SemaphoreType.DMA,
    ],
)
def _sc_agg(embcat_hbm, src2f_hbm, dst2d_hbm, ea_hbm, znd_hbm,
            aggcat_hbm,
            acc, srcbuf, dstbuf, eabuf, rows0, rows1, sem):
    c = lax.axis_index("c")
    s = lax.axis_index("s")

    pltpu.sync_copy(znd_hbm.at[pl.ds(s * NPT16, NPT16)],
                    acc.at[pl.ds(s * NPT16, NPT16)])
    plsc.subcore_barrier()

    def sg_body(sg, _):
        base = pl.multiple_of(s * EPT16 + sg * 2048, 1024)
        pltpu.sync_copy(src2f_hbm.at[pl.ds(c * E_PAD + base, 2048)], srcbuf)
        pltpu.sync_copy(ea_hbm.at[pl.ds(base, 2048)], eabuf)
        pltpu.sync_copy(dst2d_hbm.at[pl.ds(pl.multiple_of(base // 128, 8), 16)], dstbuf)

        def fire(g, buf):
            pltpu.async_copy(embcat_hbm.at[srcbuf.at[pl.ds(g * 128, 128)]],
                             buf, sem)

        def consume(g, buf):
            pltpu.make_async_copy(
                embcat_hbm.at[srcbuf.at[pl.ds(0, 128)]], buf, sem).wait()

            def e_body(e16, _):
                w16 = eabuf[pl.ds(g * 128 + e16 * 16, 16)]
                for u in range(16):
                    e = e16 * 16 + u
                    w = w16[u]
                    buf[e, pl.ds(0, 16)] = buf[e, pl.ds(0, 16)] * w
                    buf[e, pl.ds(16, 16)] = buf[e, pl.ds(16, 16)] * w
                return 0
            lax.fori_loop(0, 8, e_body, 0)
            pltpu.sync_copy(buf, acc.at[dstbuf.at[g]], add=True)

        fire(0, rows0)

        def pair_body(k, _):
            fire(2 * k + 1, rows1)
            consume(2 * k, rows0)

            @pl.when(k < 7)
            def _():
                fire(2 * k + 2, rows0)
            consume(2 * k + 1, rows1)
            return 0
        lax.fori_loop(0, 8, pair_body, 0)
        return 0
    lax.fori_loop(0, 25, sg_body, 0)
    plsc.subcore_barrier()

    pltpu.sync_copy(acc.at[pl.ds(s * NPT16, NPT16)],
                    aggcat_hbm.at[pl.ds(c * N_PAD + s * NPT16, NPT16)])


@functools.partial(
    pl.kernel,
    out_type=jax.ShapeDtypeStruct((E_PAD, 64), jnp.float32),
    mesh=_MESH,
    compiler_params=pltpu.CompilerParams(use_tc_tiling_on_sc=False),
    scratch_types=[
        pltpu.VMEM((640,), jnp.int32),
        pltpu.VMEM((640,), jnp.int32),
        pltpu.VMEM((640, 64), jnp.float32),
        pltpu.VMEM((640, 64), jnp.float32),
        pltpu.SemaphoreType.DMA,
        pltpu.SemaphoreType.DMA,
    ],
)
def _sc_decode(ab_hbm, srca_hbm, dstb_hbm,
               u_hbm,
               iabuf, ibbuf, bufa, bufb, sema, semb):
    c = lax.axis_index("c")
    s = lax.axis_index("s")
    wid = s * NC + c
    ebase = pl.multiple_of(wid * EPT32, 512)

    def it_body(it, _):
        eoff = pl.multiple_of(ebase + it * 640, 128)
        pltpu.sync_copy(srca_hbm.at[pl.ds(eoff, 640)], iabuf)
        pltpu.sync_copy(dstb_hbm.at[pl.ds(eoff, 640)], ibbuf)
        cps = []
        for j in range(5):
            cps.append(pltpu.async_copy(
                ab_hbm.at[iabuf.at[pl.ds(j * 128, 128)]],
                bufa.at[pl.ds(j * 128, 128)], sema))
            cps.append(pltpu.async_copy(
                ab_hbm.at[ibbuf.at[pl.ds(j * 128, 128)]],
                bufb.at[pl.ds(j * 128, 128)], semb))
        for cp in cps:
            cp.wait()

        def r_body(r, _):
            for q in range(4):
                va = bufa[r, pl.ds(q * 16, 16)]
                vb = bufb[r, pl.ds(q * 16, 16)]
                bufa[r, pl.ds(q * 16, 16)] = jnp.maximum(
                    va + vb, jnp.float32(0.0))
            return 0
        lax.fori_loop(0, 640, r_body, 0)
        pltpu.sync_copy(bufa, u_hbm.at[pl.ds(eoff, 640)])
        return 0
    lax.fori_loop(0, 40, it_body, 0)


def _elu(v):
    return jnp.where(v > 0, v, jnp.exp(jnp.minimum(v, 0.0)) - 1.0)


def _tc_dense_body(agglo_ref, agghi_ref, dis_ref, win_ref, bin_ref,
                   wlin_ref, blin_ref, w1_ref, b1_ref, a_ref, b_ref):
    dis = dis_ref[...]
    agg = jnp.concatenate([agglo_ref[...], agghi_ref[...]], axis=1) * dis
    t = _elu(jnp.dot(agg, win_ref[...],
                     preferred_element_type=jnp.float32) + bin_ref[...])
    nodes = _elu(jnp.dot(t, wlin_ref[...],
                         preferred_element_type=jnp.float32) + blin_ref[...])
    w1 = w1_ref[...]
    a_ref[...] = jnp.dot(nodes, w1[0:64, :],
                         preferred_element_type=jnp.float32) + b1_ref[...]
    b_ref[...] = jnp.dot(nodes, w1[64:128, :],
                         preferred_element_type=jnp.float32)


_BN = 2048


def _tc_dense(agg_cat, dis2, w_in, b_in2, w_lin, b_lin2, w1, b12):
    grid = (N_PAD // _BN,)
    full = lambda shape: pl.BlockSpec(shape, lambda i: (0, 0))
    return pl.pallas_call(
        _tc_dense_body,
        grid=grid,
        in_specs=[
            pl.BlockSpec((_BN, 32), lambda i: (i, 0)),
            pl.BlockSpec((_BN, 32), lambda i: (i + N_PAD // _BN, 0)),
            pl.BlockSpec((_BN, 1), lambda i: (i, 0)),
            full((EMB, HID)),
            full((1, HID)),
            full((HID, EMB)),
            full((1, EMB)),
            full((2 * EMB, EMB)),
            full((1, EMB)),
        ],
        out_specs=(
            pl.BlockSpec((_BN, 64), lambda i: (i, 0)),
            pl.BlockSpec((_BN, 64), lambda i: (i, 0)),
        ),
        out_shape=(
            jax.ShapeDtypeStruct((N_PAD, 64), jnp.float32),
            jax.ShapeDtypeStruct((N_PAD, 64), jnp.float32),
        ),
    )(agg_cat, agg_cat, dis2, w_in, b_in2, w_lin, b_lin2, w1, b12)


_BE = 4096


def _tc_mlp_body(u_ref, w2_ref, b2_ref, w3_ref, b3_ref, out_ref):
    h = jnp.maximum(jnp.dot(u_ref[...], w2_ref[...],
                            preferred_element_type=jnp.float32) + b2_ref[...],
                    0.0)
    res = jnp.dot(h, w3_ref[...].T, preferred_element_type=jnp.float32)
    out_ref[...] = res[:, 0] + b3_ref[0, 0]


def _tc_mlp(u, w2, b22, w3r, b32):
    grid = (E_PAD // _BE,)
    full = lambda shape: pl.BlockSpec(shape, lambda i: (0, 0))
    return pl.pallas_call(
        _tc_mlp_body,
        grid=grid,
        in_specs=[
            pl.BlockSpec((_BE, 64), lambda i: (i, 0)),
            full((EMB, EMB)),
            full((1, EMB)),
            full((1, EMB)),
            full((1, 1)),
        ],
        out_specs=pl.BlockSpec((_BE,), lambda i: (i,)),
        out_shape=jax.ShapeDtypeStruct((E_PAD,), jnp.float32),
    )(u, w2, b22, w3r, b32)


def kernel(x, edge_index, edge_attr, emb_table, W_in, b_in, W_lin, b_lin,
           W1, b1, W2, b2, W3, b3):
    src = edge_index[0]
    dst = edge_index[1]

    epad = E_PAD - E
    src_p = jnp.concatenate([src, jnp.zeros((epad,), jnp.int32)])
    dst_p = jnp.concatenate([dst, jnp.zeros((epad,), jnp.int32)])
    ea_p = jnp.concatenate([edge_attr, jnp.zeros((epad,), jnp.float32)])
    x_p = jnp.concatenate([x, jnp.zeros((N_PAD - N,), jnp.int32)])

    dst2d = dst_p.reshape(E_PAD // 128, 128)
    src2f = jnp.concatenate([src_p, src_p + N_PAD])
    dstb = dst_p + N_PAD

    zn = jnp.zeros((N_PAD,), jnp.float32)
    znd = jnp.zeros((N_PAD, 32), jnp.float32)

    dis, emb_cat = _sc_prep(emb_table, x_p, dst2d, ea_p, zn)
    agg_cat = _sc_agg(emb_cat, src2f, dst2d, ea_p, znd)

    a_mat, b_mat = _tc_dense(
        agg_cat, dis.reshape(N_PAD, 1),
        W_in, b_in.reshape(1, HID),
        W_lin, b_lin.reshape(1, EMB),
        W1, b1.reshape(1, EMB))

    ab = jnp.concatenate([a_mat, b_mat], axis=0)
    u = _sc_decode(ab, src_p, dstb)

    out = _tc_mlp(u, W2, b2.reshape(1, EMB),
                  W3.reshape(1, EMB), b3.reshape(1, 1))
    return out[:E]


# speedup vs baseline: 7.6627x; 1.0212x over previous
import functools

import jax
import jax.numpy as jnp
from jax import lax
from jax.experimental import pallas as pl
from jax.experimental.pallas import tpu as pltpu
from jax.experimental.pallas import tpu_sc as plsc

N = 50000
E = 800000
EMB = 64
HID = 128

NC = 2
NS = 16
TILES = NC * NS

N_PAD = 53248
E_PAD = 819200

NPT32 = N_PAD // TILES
NPT16 = N_PAD // NS
EPT16 = E_PAD // NS
EPT32 = E_PAD // TILES

_MESH = plsc.VectorSubcoreMesh(core_axis_name="c", subcore_axis_name="s")


def _rsqrt16(v):
    i = lax.bitcast_convert_type(v, jnp.int32)
    i = jnp.int32(0x5F3759DF) - (i >> 1)
    y = lax.bitcast_convert_type(i, jnp.float32)
    for _ in range(3):
        y = y * (jnp.float32(1.5) - jnp.float32(0.5) * v * y * y)
    return jnp.where(v > jnp.float32(0.0), y, jnp.float32(0.0))


@functools.partial(
    pl.kernel,
    out_type=(
        jax.ShapeDtypeStruct((N_PAD,), jnp.float32),
        jax.ShapeDtypeStruct((2 * N_PAD, 32), jnp.float32),
    ),
    mesh=_MESH,
    compiler_params=pltpu.CompilerParams(use_tc_tiling_on_sc=False),
    scratch_types=[
        pltpu.VMEM_SHARED((N_PAD,), jnp.float32),
        pltpu.VMEM((10240,), jnp.float32),
        pltpu.VMEM((80, 128), jnp.int32),
        pltpu.VMEM((NPT32,), jnp.float32),
        pltpu.VMEM((NPT32,), jnp.float32),
        pltpu.VMEM((NPT32,), jnp.int32),
        pltpu.VMEM((128, 64), jnp.float32),
        pltpu.VMEM((128, 32), jnp.float32),
        pltpu.VMEM((128, 32), jnp.float32),
        pltpu.SemaphoreType.DMA,
    ],
)
def _sc_prep(emb_hbm, x_hbm, dst2d_hbm, ea_hbm, zn_hbm,
             dis_hbm, embcat_hbm,
             deg_acc, eabuf, dstbuf, degbuf, disbuf, xbuf, rows, lobuf, hibuf,
             sem):
    c = lax.axis_index("c")
    s = lax.axis_index("s")
    wid = s * NC + c

    pltpu.sync_copy(zn_hbm.at[pl.ds(s * NPT16, NPT16)],
                    deg_acc.at[pl.ds(s * NPT16, NPT16)])
    plsc.subcore_barrier()

    def sg_body(sg, _):
        base = pl.multiple_of(s * EPT16 + sg * 10240, 1024)
        pltpu.sync_copy(ea_hbm.at[pl.ds(base, 10240)], eabuf)
        pltpu.sync_copy(dst2d_hbm.at[pl.ds(pl.multiple_of(base // 128, 8), 80)], dstbuf)

        def g_fire(g, _):
            pltpu.async_copy(eabuf.at[pl.ds(g * 128, 128)],
                             deg_acc.at[dstbuf.at[g]], sem, add=True)
            return 0
        lax.fori_loop(0, 80, g_fire, 0)

        def g_drain(g, _):
            pltpu.make_async_copy(eabuf.at[pl.ds(0, 128)],
                                  deg_acc.at[dstbuf.at[0]], sem).wait()
            return 0
        lax.fori_loop(0, 80, g_drain, 0)
        return 0
    lax.fori_loop(0, 5, sg_body, 0)
    plsc.subcore_barrier()

    nbase = pl.multiple_of(wid * NPT32, 128)
    pltpu.sync_copy(deg_acc.at[pl.ds(nbase, NPT32)], degbuf)
    pltpu.sync_copy(x_hbm.at[pl.ds(nbase, NPT32)], xbuf)

    def dis_body(i, _):
        disbuf[pl.ds(i * 16, 16)] = _rsqrt16(degbuf[pl.ds(i * 16, 16)])
        return 0
    lax.fori_loop(0, NPT32 // 16, dis_body, 0)
    pltpu.sync_copy(disbuf, dis_hbm.at[pl.ds(nbase, NPT32)])

    def j_body(j, _):
        pltpu.async_copy(emb_hbm.at[xbuf.at[pl.ds(j * 128, 128)]],
                         rows, sem).wait()

        def e_body(e16, _):
            w16 = disbuf[pl.ds(j * 128 + e16 * 16, 16)]
            for u in range(16):
                e = e16 * 16 + u
                w = w16[u]
                lobuf[e, pl.ds(0, 16)] = rows[e, pl.ds(0, 16)] * w
                lobuf[e, pl.ds(16, 16)] = rows[e, pl.ds(16, 16)] * w
                hibuf[e, pl.ds(0, 16)] = rows[e, pl.ds(32, 16)] * w
                hibuf[e, pl.ds(16, 16)] = rows[e, pl.ds(48, 16)] * w
            return 0
        lax.fori_loop(0, 8, e_body, 0)
        pltpu.sync_copy(lobuf, embcat_hbm.at[pl.ds(nbase + j * 128, 128)])
        pltpu.sync_copy(hibuf, embcat_hbm.at[pl.ds(N_PAD + nbase + j * 128, 128)])
        return 0
    lax.fori_loop(0, 13, j_body, 0)


@functools.partial(
    pl.kernel,
    out_type=jax.ShapeDtypeStruct((2 * N_PAD, 32), jnp.float32),
    mesh=_MESH,
    compiler_params=pltpu.CompilerParams(use_tc_tiling_on_sc=False),
    scratch_types=[
        pltpu.VMEM_SHARED((N_PAD, 32), jnp.float32),
        pltpu.VMEM((2048,), jnp.int32),
        pltpu.VMEM((16, 128), jnp.int32),
        pltpu.VMEM((2048,), jnp.float32),
        pltpu.VMEM((128, 32), jnp.float32),
        pltpu.VMEM((128, 32), jnp.float32),
        pltpu.SemaphoreType.DMA,
        pltpu.SemaphoreType.DMA,
        pltpu.SemaphoreType.DMA,
        pltpu.SemaphoreType.DMA,
    ],
)
def _sc_agg(embcat_hbm, src2f_hbm, dst2d_hbm, ea_hbm, znd_hbm,
            aggcat_hbm,
            acc, srcbuf, dstbuf, eabuf, rows0, rows1, sg0, sg1, ss0, ss1):
    c = lax.axis_index("c")
    s = lax.axis_index("s")

    pltpu.sync_copy(znd_hbm.at[pl.ds(s * NPT16, NPT16)],
                    acc.at[pl.ds(s * NPT16, NPT16)])
    plsc.subcore_barrier()

    def sg_body(sg, _):
        base = pl.multiple_of(s * EPT16 + sg * 2048, 1024)
        pltpu.sync_copy(src2f_hbm.at[pl.ds(c * E_PAD + base, 2048)], srcbuf)
        pltpu.sync_copy(ea_hbm.at[pl.ds(base, 2048)], eabuf)
        pltpu.sync_copy(dst2d_hbm.at[pl.ds(pl.multiple_of(base // 128, 8), 16)], dstbuf)

        def fire(g, buf, sem):
            pltpu.async_copy(embcat_hbm.at[srcbuf.at[pl.ds(g * 128, 128)]],
                             buf, sem)

        def wait_gather(buf, sem):
            pltpu.make_async_copy(
                embcat_hbm.at[srcbuf.at[pl.ds(0, 128)]], buf, sem).wait()

        def wait_scat(buf, sem):
            pltpu.make_async_copy(buf, acc.at[dstbuf.at[0]], sem).wait()

        def scale(g, buf):
            def e_body(e16, _):
                w16 = eabuf[pl.ds(g * 128 + e16 * 16, 16)]
                for u in range(16):
                    e = e16 * 16 + u
                    w = w16[u]
                    buf[e, pl.ds(0, 16)] = buf[e, pl.ds(0, 16)] * w
                    buf[e, pl.ds(16, 16)] = buf[e, pl.ds(16, 16)] * w
                return 0
            lax.fori_loop(0, 8, e_body, 0)

        fire(0, rows0, sg0)

        def pair_body(k, _):
            @pl.when(k > 0)
            def _():
                wait_scat(rows1, ss1)
            fire(2 * k + 1, rows1, sg1)
            wait_gather(rows0, sg0)
            scale(2 * k, rows0)
            pltpu.async_copy(rows0, acc.at[dstbuf.at[2 * k]], ss0, add=True)
            wait_gather(rows1, sg1)
            scale(2 * k + 1, rows1)
            pltpu.async_copy(rows1, acc.at[dstbuf.at[2 * k + 1]], ss1,
                             add=True)
            wait_scat(rows0, ss0)

            @pl.when(k < 7)
            def _():
                fire(2 * k + 2, rows0, sg0)
            return 0
        lax.fori_loop(0, 8, pair_body, 0)
        wait_scat(rows1, ss1)
        return 0
    lax.fori_loop(0, 25, sg_body, 0)
    plsc.subcore_barrier()

    pltpu.sync_copy(acc.at[pl.ds(s * NPT16, NPT16)],
                    aggcat_hbm.at[pl.ds(c * N_PAD + s * NPT16, NPT16)])


@functools.partial(
    pl.kernel,
    out_type=jax.ShapeDtypeStruct((E_PAD, 64), jnp.float32),
    mesh=_MESH,
    compiler_params=pltpu.CompilerParams(use_tc_tiling_on_sc=False),
    scratch_types=[
        pltpu.VMEM((2560,), jnp.int32),
        pltpu.VMEM((2560,), jnp.int32),
        pltpu.VMEM((256, 64), jnp.float32),
        pltpu.VMEM((256, 64), jnp.float32),
        pltpu.VMEM((256, 64), jnp.float32),
        pltpu.VMEM((256, 64), jnp.float32),
        pltpu.SemaphoreType.DMA,
        pltpu.SemaphoreType.DMA,
        pltpu.SemaphoreType.DMA,
        pltpu.SemaphoreType.DMA,
    ],
)
def _sc_decode(ab_hbm, srca_hbm, dstb_hbm,
               u_hbm,
               iabig, ibbig, a0, b0, a1, b1, sg0, sg1, sw0, sw1):
    c = lax.axis_index("c")
    s = lax.axis_index("s")
    wid = s * NC + c
    ebase = pl.multiple_of(wid * EPT32, 512)

    def fire_g(goff, abuf, bbuf, sem):
        for j in range(2):
            pltpu.async_copy(
                ab_hbm.at[iabig.at[pl.ds(goff + j * 128, 128)]],
                abuf.at[pl.ds(j * 128, 128)], sem)
            pltpu.async_copy(
                ab_hbm.at[ibbig.at[pl.ds(goff + j * 128, 128)]],
                bbuf.at[pl.ds(j * 128, 128)], sem)

    def wait_g(abuf, bbuf, sem):
        for j in range(2):
            pltpu.make_async_copy(
                ab_hbm.at[iabig.at[pl.ds(0, 128)]],
                abuf.at[pl.ds(j * 128, 128)], sem).wait()
            pltpu.make_async_copy(
                ab_hbm.at[ibbig.at[pl.ds(0, 128)]],
                bbuf.at[pl.ds(j * 128, 128)], sem).wait()

    def relu_add(abuf, bbuf):
        def r_body(r, _):
            for q in range(4):
                va = abuf[r, pl.ds(q * 16, 16)]
                vb = bbuf[r, pl.ds(q * 16, 16)]
                abuf[r, pl.ds(q * 16, 16)] = jnp.maximum(
                    va + vb, jnp.float32(0.0))
            return 0
        lax.fori_loop(0, 256, r_body, 0)

    def wait_w(abuf, sem):
        pltpu.make_async_copy(abuf, u_hbm.at[pl.ds(ebase, 256)], sem).wait()

    def m_body(m, _):
        moff = pl.multiple_of(ebase + m * 2560, 256)
        pltpu.sync_copy(srca_hbm.at[pl.ds(moff, 2560)], iabig)
        pltpu.sync_copy(dstb_hbm.at[pl.ds(moff, 2560)], ibbig)
        fire_g(0, a0, b0, sg0)

        def t_body(t, _):
            @pl.when(t > 0)
            def _():
                wait_w(a1, sw1)
            fire_g(t * 512 + 256, a1, b1, sg1)
            wait_g(a0, b0, sg0)
            relu_add(a0, b0)
            pltpu.async_copy(a0, u_hbm.at[pl.ds(moff + t * 512, 256)], sw0)
            wait_g(a1, b1, sg1)
            relu_add(a1, b1)
            pltpu.async_copy(
                a1, u_hbm.at[pl.ds(moff + t * 512 + 256, 256)], sw1)
            wait_w(a0, sw0)

            @pl.when(t < 4)
            def _():
                fire_g(t * 512 + 512, a0, b0, sg0)
            return 0
        lax.fori_loop(0, 5, t_body, 0)
        wait_w(a1, sw1)
        return 0
    lax.fori_loop(0, 10, m_body, 0)


def _elu(v):
    return jnp.where(v > 0, v, jnp.exp(jnp.minimum(v, 0.0)) - 1.0)


def _tc_dense_body(agglo_ref, agghi_ref, dis_ref, win_ref, bin_ref,
                   wlin_ref, blin_ref, w1_ref, b1_ref, a_ref, b_ref):
    dis = dis_ref[...]
    agg = jnp.concatenate([agglo_ref[...], agghi_ref[...]], axis=1) * dis
    t = _elu(jnp.dot(agg, win_ref[...],
                     preferred_element_type=jnp.float32) + bin_ref[...])
    nodes = _elu(jnp.dot(t, wlin_ref[...],
                         preferred_element_type=jnp.float32) + blin_ref[...])
    w1 = w1_ref[...]
    a_ref[...] = jnp.dot(nodes, w1[0:64, :],
                         preferred_element_type=jnp.float32) + b1_ref[...]
    b_ref[...] = jnp.dot(nodes, w1[64:128, :],
                         preferred_element_type=jnp.float32)


_BN = 2048


def _tc_dense(agg_cat, dis2, w_in, b_in2, w_lin, b_lin2, w1, b12):
    grid = (N_PAD // _BN,)
    full = lambda shape: pl.BlockSpec(shape, lambda i: (0, 0))
    return pl.pallas_call(
        _tc_dense_body,
        grid=grid,
        in_specs=[
            pl.BlockSpec((_BN, 32), lambda i: (i, 0)),
            pl.BlockSpec((_BN, 32), lambda i: (i + N_PAD // _BN, 0)),
            pl.BlockSpec((_BN, 1), lambda i: (i, 0)),
            full((EMB, HID)),
            full((1, HID)),
            full((HID, EMB)),
            full((1, EMB)),
            full((2 * EMB, EMB)),
            full((1, EMB)),
        ],
        out_specs=(
            pl.BlockSpec((_BN, 64), lambda i: (i, 0)),
            pl.BlockSpec((_BN, 64), lambda i: (i, 0)),
        ),
        out_shape=(
            jax.ShapeDtypeStruct((N_PAD, 64), jnp.float32),
            jax.ShapeDtypeStruct((N_PAD, 64), jnp.float32),
        ),
    )(agg_cat, agg_cat, dis2, w_in, b_in2, w_lin, b_lin2, w1, b12)


_BE = 4096


def _tc_mlp_body(u_ref, w2_ref, b2_ref, w3_ref, b3_ref, out_ref):
    h = jnp.maximum(jnp.dot(u_ref[...], w2_ref[...],
                            preferred_element_type=jnp.float32) + b2_ref[...],
                    0.0)
    res = jnp.dot(h, w3_ref[...].T, preferred_element_type=jnp.float32)
    out_ref[...] = res[:, 0] + b3_ref[0, 0]


def _tc_mlp(u, w2, b22, w3r, b32):
    grid = (E_PAD // _BE,)
    full = lambda shape: pl.BlockSpec(shape, lambda i: (0, 0))
    return pl.pallas_call(
        _tc_mlp_body,
        grid=grid,
        in_specs=[
            pl.BlockSpec((_BE, 64), lambda i: (i, 0)),
            full((EMB, EMB)),
            full((1, EMB)),
            full((1, EMB)),
            full((1, 1)),
        ],
        out_specs=pl.BlockSpec((_BE,), lambda i: (i,)),
        out_shape=jax.ShapeDtypeStruct((E_PAD,), jnp.float32),
    )(u, w2, b22, w3r, b32)


def kernel(x, edge_index, edge_attr, emb_table, W_in, b_in, W_lin, b_lin,
           W1, b1, W2, b2, W3, b3):
    src = edge_index[0]
    dst = edge_index[1]

    epad = E_PAD - E
    src_p = jnp.concatenate([src, jnp.zeros((epad,), jnp.int32)])
    dst_p = jnp.concatenate([dst, jnp.zeros((epad,), jnp.int32)])
    ea_p = jnp.concatenate([edge_attr, jnp.zeros((epad,), jnp.float32)])
    x_p = jnp.concatenate([x, jnp.zeros((N_PAD - N,), jnp.int32)])

    dst2d = dst_p.reshape(E_PAD // 128, 128)
    src2f = jnp.concatenate([src_p, src_p + N_PAD])
    dstb = dst_p + N_PAD

    zn = jnp.zeros((N_PAD,), jnp.float32)
    znd = jnp.zeros((N_PAD, 32), jnp.float32)

    dis, emb_cat = _sc_prep(emb_table, x_p, dst2d, ea_p, zn)
    agg_cat = _sc_agg(emb_cat, src2f, dst2d, ea_p, znd)

    a_mat, b_mat = _tc_dense(
        agg_cat, dis.reshape(N_PAD, 1),
        W_in, b_in.reshape(1, HID),
        W_lin, b_lin.reshape(1, EMB),
        W1, b1.reshape(1, EMB))

    ab = jnp.concatenate([a_mat, b_mat], axis=0)
    u = _sc_decode(ab, src_p, dstb)

    out = _tc_mlp(u, W2, b2.reshape(1, EMB),
                  W3.reshape(1, EMB), b3.reshape(1, 1))
    return out[:E]


# speedup vs baseline: 7.7921x; 1.0169x over previous
import functools

import jax
import jax.numpy as jnp
from jax import lax
from jax.experimental import pallas as pl
from jax.experimental.pallas import tpu as pltpu
from jax.experimental.pallas import tpu_sc as plsc

N = 50000
E = 800000
EMB = 64
HID = 128

NC = 2
NS = 16
TILES = NC * NS

N_PAD = 53248
E_PAD = 819200

NPT32 = N_PAD // TILES
NPT16 = N_PAD // NS
EPT16 = E_PAD // NS
EPT32 = E_PAD // TILES

_MESH = plsc.VectorSubcoreMesh(core_axis_name="c", subcore_axis_name="s")


def _rsqrt16(v):
    i = lax.bitcast_convert_type(v, jnp.int32)
    i = jnp.int32(0x5F3759DF) - (i >> 1)
    y = lax.bitcast_convert_type(i, jnp.float32)
    for _ in range(3):
        y = y * (jnp.float32(1.5) - jnp.float32(0.5) * v * y * y)
    return jnp.where(v > jnp.float32(0.0), y, jnp.float32(0.0))


@functools.partial(
    pl.kernel,
    out_type=(
        jax.ShapeDtypeStruct((N_PAD,), jnp.float32),
        jax.ShapeDtypeStruct((2 * N_PAD, 32), jnp.float32),
    ),
    mesh=_MESH,
    compiler_params=pltpu.CompilerParams(use_tc_tiling_on_sc=False),
    scratch_types=[
        pltpu.VMEM_SHARED((N_PAD,), jnp.float32),
        pltpu.VMEM((10240,), jnp.float32),
        pltpu.VMEM((80, 128), jnp.int32),
        pltpu.VMEM((NPT32,), jnp.float32),
        pltpu.VMEM((NPT32,), jnp.float32),
        pltpu.VMEM((NPT32,), jnp.int32),
        pltpu.VMEM((128, 64), jnp.float32),
        pltpu.VMEM((128, 32), jnp.float32),
        pltpu.VMEM((128, 32), jnp.float32),
        pltpu.SemaphoreType.DMA,
    ],
)
def _sc_prep(emb_hbm, x_hbm, dst2d_hbm, ea_hbm, zn_hbm,
             dis_hbm, embcat_hbm,
             deg_acc, eabuf, dstbuf, degbuf, disbuf, xbuf, rows, lobuf, hibuf,
             sem):
    c = lax.axis_index("c")
    s = lax.axis_index("s")
    wid = s * NC + c

    pltpu.sync_copy(zn_hbm.at[pl.ds(s * NPT16, NPT16)],
                    deg_acc.at[pl.ds(s * NPT16, NPT16)])
    plsc.subcore_barrier()

    def sg_body(sg, _):
        base = pl.multiple_of(s * EPT16 + sg * 10240, 1024)
        pltpu.sync_copy(ea_hbm.at[pl.ds(base, 10240)], eabuf)
        pltpu.sync_copy(dst2d_hbm.at[pl.ds(pl.multiple_of(base // 128, 8), 80)], dstbuf)

        def g_fire(g, _):
            pltpu.async_copy(eabuf.at[pl.ds(g * 128, 128)],
                             deg_acc.at[dstbuf.at[g]], sem, add=True)
            return 0
        lax.fori_loop(0, 80, g_fire, 0)

        def g_drain(g, _):
            pltpu.make_async_copy(eabuf.at[pl.ds(0, 128)],
                                  deg_acc.at[dstbuf.at[0]], sem).wait()
            return 0
        lax.fori_loop(0, 80, g_drain, 0)
        return 0
    lax.fori_loop(0, 5, sg_body, 0)
    plsc.subcore_barrier()

    nbase = pl.multiple_of(wid * NPT32, 128)
    pltpu.sync_copy(deg_acc.at[pl.ds(nbase, NPT32)], degbuf)
    pltpu.sync_copy(x_hbm.at[pl.ds(nbase, NPT32)], xbuf)

    def dis_body(i, _):
        disbuf[pl.ds(i * 16, 16)] = _rsqrt16(degbuf[pl.ds(i * 16, 16)])
        return 0
    lax.fori_loop(0, NPT32 // 16, dis_body, 0)
    pltpu.sync_copy(disbuf, dis_hbm.at[pl.ds(nbase, NPT32)])

    def j_body(j, _):
        pltpu.async_copy(emb_hbm.at[xbuf.at[pl.ds(j * 128, 128)]],
                         rows, sem).wait()

        def e_body(e16, _):
            w16 = disbuf[pl.ds(j * 128 + e16 * 16, 16)]
            for u in range(16):
                e = e16 * 16 + u
                w = w16[u]
                lobuf[e, pl.ds(0, 16)] = rows[e, pl.ds(0, 16)] * w
                lobuf[e, pl.ds(16, 16)] = rows[e, pl.ds(16, 16)] * w
                hibuf[e, pl.ds(0, 16)] = rows[e, pl.ds(32, 16)] * w
                hibuf[e, pl.ds(16, 16)] = rows[e, pl.ds(48, 16)] * w
            return 0
        lax.fori_loop(0, 8, e_body, 0)
        pltpu.sync_copy(lobuf, embcat_hbm.at[pl.ds(nbase + j * 128, 128)])
        pltpu.sync_copy(hibuf, embcat_hbm.at[pl.ds(N_PAD + nbase + j * 128, 128)])
        return 0
    lax.fori_loop(0, 13, j_body, 0)


@functools.partial(
    pl.kernel,
    out_type=jax.ShapeDtypeStruct((2 * N_PAD, 32), jnp.float32),
    mesh=_MESH,
    compiler_params=pltpu.CompilerParams(use_tc_tiling_on_sc=False),
    scratch_types=[
        pltpu.VMEM_SHARED((N_PAD, 32), jnp.float32),
        pltpu.VMEM((2048,), jnp.int32),
        pltpu.VMEM((16, 128), jnp.int32),
        pltpu.VMEM((2048,), jnp.float32),
        pltpu.VMEM((128, 32), jnp.float32),
        pltpu.VMEM((128, 32), jnp.float32),
        pltpu.SemaphoreType.DMA,
        pltpu.SemaphoreType.DMA,
        pltpu.SemaphoreType.DMA,
        pltpu.SemaphoreType.DMA,
    ],
)
def _sc_agg(embcat_hbm, src2f_hbm, dst2d_hbm, ea_hbm, znd_hbm,
            aggcat_hbm,
            acc, srcbuf, dstbuf, eabuf, rows0, rows1, sg0, sg1, ss0, ss1):
    c = lax.axis_index("c")
    s = lax.axis_index("s")

    pltpu.sync_copy(znd_hbm.at[pl.ds(s * NPT16, NPT16)],
                    acc.at[pl.ds(s * NPT16, NPT16)])
    plsc.subcore_barrier()

    def sg_body(sg, _):
        base = pl.multiple_of(s * EPT16 + sg * 2048, 1024)
        pltpu.sync_copy(src2f_hbm.at[pl.ds(c * E_PAD + base, 2048)], srcbuf)
        pltpu.sync_copy(ea_hbm.at[pl.ds(base, 2048)], eabuf)
        pltpu.sync_copy(dst2d_hbm.at[pl.ds(pl.multiple_of(base // 128, 8), 16)], dstbuf)

        def fire(g, buf, sem):
            pltpu.async_copy(embcat_hbm.at[srcbuf.at[pl.ds(g * 128, 128)]],
                             buf, sem)

        def wait_gather(buf, sem):
            pltpu.make_async_copy(
                embcat_hbm.at[srcbuf.at[pl.ds(0, 128)]], buf, sem).wait()

        def wait_scat(buf, sem):
            pltpu.make_async_copy(buf, acc.at[dstbuf.at[0]], sem).wait()

        def scale(g, buf):
            def e_body(e16, _):
                w16 = eabuf[pl.ds(g * 128 + e16 * 32, 16)]
                w16b = eabuf[pl.ds(g * 128 + e16 * 32 + 16, 16)]
                for u in range(16):
                    e16e = e16 * 32 + u
                    w = w16[u]
                    buf[e16e, pl.ds(0, 16)] = buf[e16e, pl.ds(0, 16)] * w
                    buf[e16e, pl.ds(16, 16)] = buf[e16e, pl.ds(16, 16)] * w
                for u in range(16):
                    e16e = e16 * 32 + 16 + u
                    w = w16b[u]
                    buf[e16e, pl.ds(0, 16)] = buf[e16e, pl.ds(0, 16)] * w
                    buf[e16e, pl.ds(16, 16)] = buf[e16e, pl.ds(16, 16)] * w
                return 0
            lax.fori_loop(0, 4, e_body, 0)

        fire(0, rows0, sg0)

        def pair_body(k, _):
            @pl.when(k > 0)
            def _():
                wait_scat(rows1, ss1)
            fire(2 * k + 1, rows1, sg1)
            wait_gather(rows0, sg0)
            scale(2 * k, rows0)
            pltpu.async_copy(rows0, acc.at[dstbuf.at[2 * k]], ss0, add=True)
            wait_gather(rows1, sg1)
            scale(2 * k + 1, rows1)
            pltpu.async_copy(rows1, acc.at[dstbuf.at[2 * k + 1]], ss1,
                             add=True)
            wait_scat(rows0, ss0)

            @pl.when(k < 7)
            def _():
                fire(2 * k + 2, rows0, sg0)
            return 0
        lax.fori_loop(0, 8, pair_body, 0)
        wait_scat(rows1, ss1)
        return 0
    lax.fori_loop(0, 25, sg_body, 0)
    plsc.subcore_barrier()

    pltpu.sync_copy(acc.at[pl.ds(s * NPT16, NPT16)],
                    aggcat_hbm.at[pl.ds(c * N_PAD + s * NPT16, NPT16)])


@functools.partial(
    pl.kernel,
    out_type=jax.ShapeDtypeStruct((E_PAD, 64), jnp.float32),
    mesh=_MESH,
    compiler_params=pltpu.CompilerParams(use_tc_tiling_on_sc=False),
    scratch_types=[
        pltpu.VMEM((2560,), jnp.int32),
        pltpu.VMEM((2560,), jnp.int32),
        pltpu.VMEM((256, 64), jnp.float32),
        pltpu.VMEM((256, 64), jnp.float32),
        pltpu.VMEM((256, 64), jnp.float32),
        pltpu.VMEM((256, 64), jnp.float32),
        pltpu.SemaphoreType.DMA,
        pltpu.SemaphoreType.DMA,
        pltpu.SemaphoreType.DMA,
        pltpu.SemaphoreType.DMA,
    ],
)
def _sc_decode(ab_hbm, srca_hbm, dstb_hbm,
               u_hbm,
               iabig, ibbig, a0, b0, a1, b1, sg0, sg1, sw0, sw1):
    c = lax.axis_index("c")
    s = lax.axis_index("s")
    wid = s * NC + c
    ebase = pl.multiple_of(wid * EPT32, 512)

    def fire_g(goff, abuf, bbuf, sem):
        for j in range(2):
            pltpu.async_copy(
                ab_hbm.at[iabig.at[pl.ds(goff + j * 128, 128)]],
                abuf.at[pl.ds(j * 128, 128)], sem)
            pltpu.async_copy(
                ab_hbm.at[ibbig.at[pl.ds(goff + j * 128, 128)]],
                bbuf.at[pl.ds(j * 128, 128)], sem)

    def wait_g(abuf, bbuf, sem):
        for j in range(2):
            pltpu.make_async_copy(
                ab_hbm.at[iabig.at[pl.ds(0, 128)]],
                abuf.at[pl.ds(j * 128, 128)], sem).wait()
            pltpu.make_async_copy(
                ab_hbm.at[ibbig.at[pl.ds(0, 128)]],
                bbuf.at[pl.ds(j * 128, 128)], sem).wait()

    def relu_add(abuf, bbuf):
        def r_body(r4, _):
            for rr in range(4):
                r = r4 * 4 + rr
                for q in range(4):
                    va = abuf[r, pl.ds(q * 16, 16)]
                    vb = bbuf[r, pl.ds(q * 16, 16)]
                    abuf[r, pl.ds(q * 16, 16)] = jnp.maximum(
                        va + vb, jnp.float32(0.0))
            return 0
        lax.fori_loop(0, 64, r_body, 0)

    def wait_w(abuf, sem):
        pltpu.make_async_copy(abuf, u_hbm.at[pl.ds(ebase, 256)], sem).wait()

    def m_body(m, _):
        moff = pl.multiple_of(ebase + m * 2560, 256)
        pltpu.sync_copy(srca_hbm.at[pl.ds(moff, 2560)], iabig)
        pltpu.sync_copy(dstb_hbm.at[pl.ds(moff, 2560)], ibbig)
        fire_g(0, a0, b0, sg0)

        def t_body(t, _):
            @pl.when(t > 0)
            def _():
                wait_w(a1, sw1)
            fire_g(t * 512 + 256, a1, b1, sg1)
            wait_g(a0, b0, sg0)
            relu_add(a0, b0)
            pltpu.async_copy(a0, u_hbm.at[pl.ds(moff + t * 512, 256)], sw0)
            wait_g(a1, b1, sg1)
            relu_add(a1, b1)
            pltpu.async_copy(
                a1, u_hbm.at[pl.ds(moff + t * 512 + 256, 256)], sw1)
            wait_w(a0, sw0)

            @pl.when(t < 4)
            def _():
                fire_g(t * 512 + 512, a0, b0, sg0)
            return 0
        lax.fori_loop(0, 5, t_body, 0)
        wait_w(a1, sw1)
        return 0
    lax.fori_loop(0, 10, m_body, 0)


def _elu(v):
    return jnp.where(v > 0, v, jnp.exp(jnp.minimum(v, 0.0)) - 1.0)


def _tc_dense_body(agglo_ref, agghi_ref, dis_ref, win_ref, bin_ref,
                   wlin_ref, blin_ref, w1_ref, b1_ref, a_ref, b_ref):
    dis = dis_ref[...]
    agg = jnp.concatenate([agglo_ref[...], agghi_ref[...]], axis=1) * dis
    t = _elu(jnp.dot(agg, win_ref[...],
                     preferred_element_type=jnp.float32) + bin_ref[...])
    nodes = _elu(jnp.dot(t, wlin_ref[...],
                         preferred_element_type=jnp.float32) + blin_ref[...])
    w1 = w1_ref[...]
    a_ref[...] = jnp.dot(nodes, w1[0:64, :],
                         preferred_element_type=jnp.float32) + b1_ref[...]
    b_ref[...] = jnp.dot(nodes, w1[64:128, :],
                         preferred_element_type=jnp.float32)


_BN = 2048


def _tc_dense(agg_cat, dis2, w_in, b_in2, w_lin, b_lin2, w1, b12):
    grid = (N_PAD // _BN,)
    full = lambda shape: pl.BlockSpec(shape, lambda i: (0, 0))
    return pl.pallas_call(
        _tc_dense_body,
        grid=grid,
        in_specs=[
            pl.BlockSpec((_BN, 32), lambda i: (i, 0)),
            pl.BlockSpec((_BN, 32), lambda i: (i + N_PAD // _BN, 0)),
            pl.BlockSpec((_BN, 1), lambda i: (i, 0)),
            full((EMB, HID)),
            full((1, HID)),
            full((HID, EMB)),
            full((1, EMB)),
            full((2 * EMB, EMB)),
            full((1, EMB)),
        ],
        out_specs=(
            pl.BlockSpec((_BN, 64), lambda i: (i, 0)),
            pl.BlockSpec((_BN, 64), lambda i: (i, 0)),
        ),
        out_shape=(
            jax.ShapeDtypeStruct((N_PAD, 64), jnp.float32),
            jax.ShapeDtypeStruct((N_PAD, 64), jnp.float32),
        ),
    )(agg_cat, agg_cat, dis2, w_in, b_in2, w_lin, b_lin2, w1, b12)


_BE = 8192


def _tc_mlp_body(u_ref, w2_ref, b2_ref, w3_ref, b3_ref, out_ref):
    h = jnp.maximum(jnp.dot(u_ref[...], w2_ref[...],
                            preferred_element_type=jnp.float32) + b2_ref[...],
                    0.0)
    res = jnp.dot(h, w3_ref[...].T, preferred_element_type=jnp.float32)
    out_ref[...] = res[:, 0] + b3_ref[0, 0]


def _tc_mlp(u, w2, b22, w3r, b32):
    grid = (E_PAD // _BE,)
    full = lambda shape: pl.BlockSpec(shape, lambda i: (0, 0))
    return pl.pallas_call(
        _tc_mlp_body,
        grid=grid,
        in_specs=[
            pl.BlockSpec((_BE, 64), lambda i: (i, 0)),
            full((EMB, EMB)),
            full((1, EMB)),
            full((1, EMB)),
            full((1, 1)),
        ],
        out_specs=pl.BlockSpec((_BE,), lambda i: (i,)),
        out_shape=jax.ShapeDtypeStruct((E_PAD,), jnp.float32),
    )(u, w2, b22, w3r, b32)


def kernel(x, edge_index, edge_attr, emb_table, W_in, b_in, W_lin, b_lin,
           W1, b1, W2, b2, W3, b3):
    src = edge_index[0]
    dst = edge_index[1]

    epad = E_PAD - E
    src_p = jnp.concatenate([src, jnp.zeros((epad,), jnp.int32)])
    dst_p = jnp.concatenate([dst, jnp.zeros((epad,), jnp.int32)])
    ea_p = jnp.concatenate([edge_attr, jnp.zeros((epad,), jnp.float32)])
    x_p = jnp.concatenate([x, jnp.zeros((N_PAD - N,), jnp.int32)])

    dst2d = dst_p.reshape(E_PAD // 128, 128)
    src2f = jnp.concatenate([src_p, src_p + N_PAD])
    dstb = dst_p + N_PAD

    zn = jnp.zeros((N_PAD,), jnp.float32)
    znd = jnp.zeros((N_PAD, 32), jnp.float32)

    dis, emb_cat = _sc_prep(emb_table, x_p, dst2d, ea_p, zn)
    agg_cat = _sc_agg(emb_cat, src2f, dst2d, ea_p, znd)

    a_mat, b_mat = _tc_dense(
        agg_cat, dis.reshape(N_PAD, 1),
        W_in, b_in.reshape(1, HID),
        W_lin, b_lin.reshape(1, EMB),
        W1, b1.reshape(1, EMB))

    ab = jnp.concatenate([a_mat, b_mat], axis=0)
    u = _sc_decode(ab, src_p, dstb)

    out = _tc_mlp(u, W2, b2.reshape(1, EMB),
                  W3.reshape(1, EMB), b3.reshape(1, 1))
    return out[:E]


# speedup vs baseline: 8.6235x; 1.1067x over previous
import functools

import jax
import jax.numpy as jnp
from jax import lax
from jax.experimental import pallas as pl
from jax.experimental.pallas import tpu as pltpu
from jax.experimental.pallas import tpu_sc as plsc

N = 50000
E = 800000
EMB = 64
HID = 128

NC = 2
NS = 16
TILES = NC * NS

N_PAD = 53248
E_PAD = 819200

NPT32 = N_PAD // TILES
NPT16 = N_PAD // NS
EPT16 = E_PAD // NS
EPT32 = E_PAD // TILES

_MESH = plsc.VectorSubcoreMesh(core_axis_name="c", subcore_axis_name="s")


def _rsqrt16(v):
    i = lax.bitcast_convert_type(v, jnp.int32)
    i = jnp.int32(0x5F3759DF) - (i >> 1)
    y = lax.bitcast_convert_type(i, jnp.float32)
    for _ in range(3):
        y = y * (jnp.float32(1.5) - jnp.float32(0.5) * v * y * y)
    return jnp.where(v > jnp.float32(0.0), y, jnp.float32(0.0))


@functools.partial(
    pl.kernel,
    out_type=(
        jax.ShapeDtypeStruct((N_PAD,), jnp.float32),
        jax.ShapeDtypeStruct((2 * N_PAD, 32), jnp.float32),
    ),
    mesh=_MESH,
    compiler_params=pltpu.CompilerParams(use_tc_tiling_on_sc=False),
    scratch_types=[
        pltpu.VMEM_SHARED((N_PAD,), jnp.float32),
        pltpu.VMEM((10240,), jnp.float32),
        pltpu.VMEM((80, 128), jnp.int32),
        pltpu.VMEM((NPT32,), jnp.float32),
        pltpu.VMEM((NPT32,), jnp.float32),
        pltpu.VMEM((NPT32,), jnp.int32),
        pltpu.VMEM((128, 64), jnp.float32),
        pltpu.VMEM((128, 32), jnp.float32),
        pltpu.VMEM((128, 32), jnp.float32),
        pltpu.SemaphoreType.DMA,
    ],
)
def _sc_prep(emb_hbm, x_hbm, dst2d_hbm, ea_hbm, zn_hbm,
             dis_hbm, embcat_hbm,
             deg_acc, eabuf, dstbuf, degbuf, disbuf, xbuf, rows, lobuf, hibuf,
             sem):
    c = lax.axis_index("c")
    s = lax.axis_index("s")
    wid = s * NC + c

    pltpu.sync_copy(zn_hbm.at[pl.ds(s * NPT16, NPT16)],
                    deg_acc.at[pl.ds(s * NPT16, NPT16)])
    plsc.subcore_barrier()

    def sg_body(sg, _):
        base = pl.multiple_of(s * EPT16 + sg * 10240, 1024)
        pltpu.sync_copy(ea_hbm.at[pl.ds(base, 10240)], eabuf)
        pltpu.sync_copy(dst2d_hbm.at[pl.ds(pl.multiple_of(base // 128, 8), 80)], dstbuf)

        def g_fire(g, _):
            pltpu.async_copy(eabuf.at[pl.ds(g * 128, 128)],
                             deg_acc.at[dstbuf.at[g]], sem, add=True)
            return 0
        lax.fori_loop(0, 80, g_fire, 0)

        def g_drain(g, _):
            pltpu.make_async_copy(eabuf.at[pl.ds(0, 128)],
                                  deg_acc.at[dstbuf.at[0]], sem).wait()
            return 0
        lax.fori_loop(0, 80, g_drain, 0)
        return 0
    lax.fori_loop(0, 5, sg_body, 0)
    plsc.subcore_barrier()

    nbase = pl.multiple_of(wid * NPT32, 128)
    pltpu.sync_copy(deg_acc.at[pl.ds(nbase, NPT32)], degbuf)
    pltpu.sync_copy(x_hbm.at[pl.ds(nbase, NPT32)], xbuf)

    def dis_body(i, _):
        disbuf[pl.ds(i * 16, 16)] = _rsqrt16(degbuf[pl.ds(i * 16, 16)])
        return 0
    lax.fori_loop(0, NPT32 // 16, dis_body, 0)
    pltpu.sync_copy(disbuf, dis_hbm.at[pl.ds(nbase, NPT32)])

    def j_body(j, _):
        pltpu.async_copy(emb_hbm.at[xbuf.at[pl.ds(j * 128, 128)]],
                         rows, sem).wait()

        def e_body(e16, _):
            w16 = disbuf[pl.ds(j * 128 + e16 * 16, 16)]
            for u in range(16):
                e = e16 * 16 + u
                w = w16[u]
                lobuf[e, pl.ds(0, 16)] = rows[e, pl.ds(0, 16)] * w
                lobuf[e, pl.ds(16, 16)] = rows[e, pl.ds(16, 16)] * w
                hibuf[e, pl.ds(0, 16)] = rows[e, pl.ds(32, 16)] * w
                hibuf[e, pl.ds(16, 16)] = rows[e, pl.ds(48, 16)] * w
            return 0
        lax.fori_loop(0, 8, e_body, 0)
        pltpu.sync_copy(lobuf, embcat_hbm.at[pl.ds(nbase + j * 128, 128)])
        pltpu.sync_copy(hibuf, embcat_hbm.at[pl.ds(N_PAD + nbase + j * 128, 128)])
        return 0
    lax.fori_loop(0, 13, j_body, 0)


@functools.partial(
    pl.kernel,
    out_type=jax.ShapeDtypeStruct((2 * N_PAD, 32), jnp.float32),
    mesh=_MESH,
    compiler_params=pltpu.CompilerParams(use_tc_tiling_on_sc=False),
    scratch_types=[
        pltpu.VMEM_SHARED((N_PAD, 32), jnp.float32),
        pltpu.VMEM((2048,), jnp.int32),
        pltpu.VMEM((16, 128), jnp.int32),
        pltpu.VMEM((2048,), jnp.float32),
        pltpu.VMEM((128, 32), jnp.float32),
        pltpu.VMEM((128, 32), jnp.float32),
        pltpu.SemaphoreType.DMA,
        pltpu.SemaphoreType.DMA,
        pltpu.SemaphoreType.DMA,
        pltpu.SemaphoreType.DMA,
    ],
)
def _sc_agg(embcat_hbm, src2f_hbm, dst2d_hbm, ea_hbm, znd_hbm,
            aggcat_hbm,
            acc, srcbuf, dstbuf, eabuf, rows0, rows1, sg0, sg1, ss0, ss1):
    c = lax.axis_index("c")
    s = lax.axis_index("s")

    pltpu.sync_copy(znd_hbm.at[pl.ds(s * NPT16, NPT16)],
                    acc.at[pl.ds(s * NPT16, NPT16)])
    plsc.subcore_barrier()

    def sg_body(sg, _):
        base = pl.multiple_of(s * EPT16 + sg * 2048, 1024)
        pltpu.sync_copy(src2f_hbm.at[pl.ds(c * E_PAD + base, 2048)], srcbuf)
        pltpu.sync_copy(ea_hbm.at[pl.ds(base, 2048)], eabuf)
        pltpu.sync_copy(dst2d_hbm.at[pl.ds(pl.multiple_of(base // 128, 8), 16)], dstbuf)

        def fire(g, buf, sem):
            pltpu.async_copy(embcat_hbm.at[srcbuf.at[pl.ds(g * 128, 128)]],
                             buf, sem)

        def wait_gather(buf, sem):
            pltpu.make_async_copy(
                embcat_hbm.at[srcbuf.at[pl.ds(0, 128)]], buf, sem).wait()

        def wait_scat(buf, sem):
            pltpu.make_async_copy(buf, acc.at[dstbuf.at[0]], sem).wait()

        def scale(g, buf):
            def e_body(e16, _):
                w16 = eabuf[pl.ds(g * 128 + e16 * 32, 16)]
                w16b = eabuf[pl.ds(g * 128 + e16 * 32 + 16, 16)]
                for u in range(16):
                    e16e = e16 * 32 + u
                    w = w16[u]
                    buf[e16e, pl.ds(0, 16)] = buf[e16e, pl.ds(0, 16)] * w
                    buf[e16e, pl.ds(16, 16)] = buf[e16e, pl.ds(16, 16)] * w
                for u in range(16):
                    e16e = e16 * 32 + 16 + u
                    w = w16b[u]
                    buf[e16e, pl.ds(0, 16)] = buf[e16e, pl.ds(0, 16)] * w
                    buf[e16e, pl.ds(16, 16)] = buf[e16e, pl.ds(16, 16)] * w
                return 0
            lax.fori_loop(0, 4, e_body, 0)

        fire(0, rows0, sg0)

        def pair_body(k, _):
            @pl.when(k > 0)
            def _():
                wait_scat(rows1, ss1)
            fire(2 * k + 1, rows1, sg1)
            wait_gather(rows0, sg0)
            scale(2 * k, rows0)
            pltpu.async_copy(rows0, acc.at[dstbuf.at[2 * k]], ss0, add=True)
            wait_gather(rows1, sg1)
            scale(2 * k + 1, rows1)
            pltpu.async_copy(rows1, acc.at[dstbuf.at[2 * k + 1]], ss1,
                             add=True)
            wait_scat(rows0, ss0)

            @pl.when(k < 7)
            def _():
                fire(2 * k + 2, rows0, sg0)
            return 0
        lax.fori_loop(0, 8, pair_body, 0)
        wait_scat(rows1, ss1)
        return 0
    lax.fori_loop(0, 25, sg_body, 0)
    plsc.subcore_barrier()

    pltpu.sync_copy(acc.at[pl.ds(s * NPT16, NPT16)],
                    aggcat_hbm.at[pl.ds(c * N_PAD + s * NPT16, NPT16)])


@functools.partial(
    pl.kernel,
    out_type=jax.ShapeDtypeStruct((E_PAD, 64), jnp.bfloat16),
    mesh=_MESH,
    compiler_params=pltpu.CompilerParams(use_tc_tiling_on_sc=False),
    scratch_types=[
        pltpu.VMEM((2560,), jnp.int32),
        pltpu.VMEM((2560,), jnp.int32),
        pltpu.VMEM((256, 64), jnp.bfloat16),
        pltpu.VMEM((256, 64), jnp.bfloat16),
        pltpu.VMEM((256, 64), jnp.bfloat16),
        pltpu.VMEM((256, 64), jnp.bfloat16),
        pltpu.SemaphoreType.DMA,
        pltpu.SemaphoreType.DMA,
        pltpu.SemaphoreType.DMA,
        pltpu.SemaphoreType.DMA,
    ],
)
def _sc_decode(ab_hbm, srca_hbm, dstb_hbm,
               u_hbm,
               iabig, ibbig, a0, b0, a1, b1, sg0, sg1, sw0, sw1):
    c = lax.axis_index("c")
    s = lax.axis_index("s")
    wid = s * NC + c
    ebase = pl.multiple_of(wid * EPT32, 512)

    def fire_g(goff, abuf, bbuf, sem):
        for j in range(2):
            pltpu.async_copy(
                ab_hbm.at[iabig.at[pl.ds(goff + j * 128, 128)]],
                abuf.at[pl.ds(j * 128, 128)], sem)
            pltpu.async_copy(
                ab_hbm.at[ibbig.at[pl.ds(goff + j * 128, 128)]],
                bbuf.at[pl.ds(j * 128, 128)], sem)

    def wait_g(abuf, bbuf, sem):
        for j in range(2):
            pltpu.make_async_copy(
                ab_hbm.at[iabig.at[pl.ds(0, 128)]],
                abuf.at[pl.ds(j * 128, 128)], sem).wait()
            pltpu.make_async_copy(
                ab_hbm.at[ibbig.at[pl.ds(0, 128)]],
                bbuf.at[pl.ds(j * 128, 128)], sem).wait()

    def relu_add(abuf, bbuf):
        def r_body(r4, _):
            for rr in range(4):
                r = r4 * 4 + rr
                for q in range(2):
                    va = abuf[r, pl.ds(q * 32, 32)]
                    vb = bbuf[r, pl.ds(q * 32, 32)]
                    abuf[r, pl.ds(q * 32, 32)] = jnp.maximum(va + vb, 0.0)
            return 0
        lax.fori_loop(0, 64, r_body, 0)

    def wait_w(abuf, sem):
        pltpu.make_async_copy(abuf, u_hbm.at[pl.ds(ebase, 256)], sem).wait()

    def m_body(m, _):
        moff = pl.multiple_of(ebase + m * 2560, 256)
        pltpu.sync_copy(srca_hbm.at[pl.ds(moff, 2560)], iabig)
        pltpu.sync_copy(dstb_hbm.at[pl.ds(moff, 2560)], ibbig)
        fire_g(0, a0, b0, sg0)

        def t_body(t, _):
            @pl.when(t > 0)
            def _():
                wait_w(a1, sw1)
            fire_g(t * 512 + 256, a1, b1, sg1)
            wait_g(a0, b0, sg0)
            relu_add(a0, b0)
            pltpu.async_copy(a0, u_hbm.at[pl.ds(moff + t * 512, 256)], sw0)
            wait_g(a1, b1, sg1)
            relu_add(a1, b1)
            pltpu.async_copy(
                a1, u_hbm.at[pl.ds(moff + t * 512 + 256, 256)], sw1)
            wait_w(a0, sw0)

            @pl.when(t < 4)
            def _():
                fire_g(t * 512 + 512, a0, b0, sg0)
            return 0
        lax.fori_loop(0, 5, t_body, 0)
        wait_w(a1, sw1)
        return 0
    lax.fori_loop(0, 10, m_body, 0)


def _elu(v):
    return jnp.where(v > 0, v, jnp.exp(jnp.minimum(v, 0.0)) - 1.0)


def _tc_dense_body(agglo_ref, agghi_ref, dis_ref, win_ref, bin_ref,
                   wlin_ref, blin_ref, w1_ref, b1_ref, a_ref, b_ref):
    dis = dis_ref[...]
    agg = jnp.concatenate([agglo_ref[...], agghi_ref[...]], axis=1) * dis
    t = _elu(jnp.dot(agg, win_ref[...],
                     preferred_element_type=jnp.float32) + bin_ref[...])
    nodes = _elu(jnp.dot(t, wlin_ref[...],
                         preferred_element_type=jnp.float32) + blin_ref[...])
    w1 = w1_ref[...]
    a_ref[...] = (jnp.dot(nodes, w1[0:64, :],
                          preferred_element_type=jnp.float32)
                  + b1_ref[...]).astype(jnp.bfloat16)
    b_ref[...] = jnp.dot(nodes, w1[64:128, :],
                         preferred_element_type=jnp.float32).astype(
                             jnp.bfloat16)


_BN = 2048


def _tc_dense(agg_cat, dis2, w_in, b_in2, w_lin, b_lin2, w1, b12):
    grid = (N_PAD // _BN,)
    full = lambda shape: pl.BlockSpec(shape, lambda i: (0, 0))
    return pl.pallas_call(
        _tc_dense_body,
        grid=grid,
        in_specs=[
            pl.BlockSpec((_BN, 32), lambda i: (i, 0)),
            pl.BlockSpec((_BN, 32), lambda i: (i + N_PAD // _BN, 0)),
            pl.BlockSpec((_BN, 1), lambda i: (i, 0)),
            full((EMB, HID)),
            full((1, HID)),
            full((HID, EMB)),
            full((1, EMB)),
            full((2 * EMB, EMB)),
            full((1, EMB)),
        ],
        out_specs=(
            pl.BlockSpec((_BN, 64), lambda i: (i, 0)),
            pl.BlockSpec((_BN, 64), lambda i: (i, 0)),
        ),
        out_shape=(
            jax.ShapeDtypeStruct((N_PAD, 64), jnp.bfloat16),
            jax.ShapeDtypeStruct((N_PAD, 64), jnp.bfloat16),
        ),
    )(agg_cat, agg_cat, dis2, w_in, b_in2, w_lin, b_lin2, w1, b12)


_BE = 8192


def _tc_mlp_body(u_ref, w2_ref, b2_ref, w3_ref, b3_ref, out_ref):
    h = jnp.maximum(jnp.dot(u_ref[...].astype(jnp.float32), w2_ref[...],
                            preferred_element_type=jnp.float32) + b2_ref[...],
                    0.0)
    res = jnp.dot(h, w3_ref[...].T, preferred_element_type=jnp.float32)
    out_ref[...] = res[:, 0] + b3_ref[0, 0]


def _tc_mlp(u, w2, b22, w3r, b32):
    grid = (E_PAD // _BE,)
    full = lambda shape: pl.BlockSpec(shape, lambda i: (0, 0))
    return pl.pallas_call(
        _tc_mlp_body,
        grid=grid,
        in_specs=[
            pl.BlockSpec((_BE, 64), lambda i: (i, 0)),
            full((EMB, EMB)),
            full((1, EMB)),
            full((1, EMB)),
            full((1, 1)),
        ],
        out_specs=pl.BlockSpec((_BE,), lambda i: (i,)),
        out_shape=jax.ShapeDtypeStruct((E_PAD,), jnp.float32),
    )(u, w2, b22, w3r, b32)


def kernel(x, edge_index, edge_attr, emb_table, W_in, b_in, W_lin, b_lin,
           W1, b1, W2, b2, W3, b3):
    src = edge_index[0]
    dst = edge_index[1]

    epad = E_PAD - E
    src_p = jnp.concatenate([src, jnp.zeros((epad,), jnp.int32)])
    dst_p = jnp.concatenate([dst, jnp.zeros((epad,), jnp.int32)])
    ea_p = jnp.concatenate([edge_attr, jnp.zeros((epad,), jnp.float32)])
    x_p = jnp.concatenate([x, jnp.zeros((N_PAD - N,), jnp.int32)])

    dst2d = dst_p.reshape(E_PAD // 128, 128)
    src2f = jnp.concatenate([src_p, src_p + N_PAD])
    dstb = dst_p + N_PAD

    zn = jnp.zeros((N_PAD,), jnp.float32)
    znd = jnp.zeros((N_PAD, 32), jnp.float32)

    dis, emb_cat = _sc_prep(emb_table, x_p, dst2d, ea_p, zn)
    agg_cat = _sc_agg(emb_cat, src2f, dst2d, ea_p, znd)

    a_mat, b_mat = _tc_dense(
        agg_cat, dis.reshape(N_PAD, 1),
        W_in, b_in.reshape(1, HID),
        W_lin, b_lin.reshape(1, EMB),
        W1, b1.reshape(1, EMB))

    ab = jnp.concatenate([a_mat, b_mat], axis=0)
    u = _sc_decode(ab, src_p, dstb)

    out = _tc_mlp(u, W2, b2.reshape(1, EMB),
                  W3.reshape(1, EMB), b3.reshape(1, 1))
    return out[:E]


# speedup vs baseline: 9.1825x; 1.0648x over previous
import functools

import jax
import jax.numpy as jnp
from jax import lax
from jax.experimental import pallas as pl
from jax.experimental.pallas import tpu as pltpu
from jax.experimental.pallas import tpu_sc as plsc

N = 50000
E = 800000
EMB = 64
HID = 128

NC = 2
NS = 16
TILES = NC * NS

N_PAD = 53248
E_PAD = 819200

NPT32 = N_PAD // TILES
NPT16 = N_PAD // NS
EPT16 = E_PAD // NS
EPT32 = E_PAD // TILES

_MESH = plsc.VectorSubcoreMesh(core_axis_name="c", subcore_axis_name="s")


def _rsqrt16(v):
    i = lax.bitcast_convert_type(v, jnp.int32)
    i = jnp.int32(0x5F3759DF) - (i >> 1)
    y = lax.bitcast_convert_type(i, jnp.float32)
    for _ in range(3):
        y = y * (jnp.float32(1.5) - jnp.float32(0.5) * v * y * y)
    return jnp.where(v > jnp.float32(0.0), y, jnp.float32(0.0))


@functools.partial(
    pl.kernel,
    out_type=(
        jax.ShapeDtypeStruct((N_PAD,), jnp.float32),
        jax.ShapeDtypeStruct((2 * N_PAD, 32), jnp.float32),
    ),
    mesh=_MESH,
    compiler_params=pltpu.CompilerParams(use_tc_tiling_on_sc=False),
    scratch_types=[
        pltpu.VMEM_SHARED((N_PAD,), jnp.float32),
        pltpu.VMEM((10240,), jnp.float32),
        pltpu.VMEM((80, 128), jnp.int32),
        pltpu.VMEM((NPT32,), jnp.float32),
        pltpu.VMEM((NPT32,), jnp.float32),
        pltpu.VMEM((NPT32,), jnp.int32),
        pltpu.VMEM((128, 64), jnp.float32),
        pltpu.VMEM((128, 32), jnp.float32),
        pltpu.VMEM((128, 32), jnp.float32),
        pltpu.SemaphoreType.DMA,
    ],
)
def _sc_prep(emb_hbm, x_hbm, dst2d_hbm, ea_hbm, zn_hbm,
             dis_hbm, embcat_hbm,
             deg_acc, eabuf, dstbuf, degbuf, disbuf, xbuf, rows, lobuf, hibuf,
             sem):
    c = lax.axis_index("c")
    s = lax.axis_index("s")
    wid = s * NC + c

    pltpu.sync_copy(zn_hbm.at[pl.ds(s * NPT16, NPT16)],
                    deg_acc.at[pl.ds(s * NPT16, NPT16)])
    plsc.subcore_barrier()

    def sg_body(sg, _):
        base = pl.multiple_of(s * EPT16 + sg * 10240, 1024)
        pltpu.sync_copy(ea_hbm.at[pl.ds(base, 10240)], eabuf)
        pltpu.sync_copy(dst2d_hbm.at[pl.ds(pl.multiple_of(base // 128, 8), 80)], dstbuf)

        def g_fire(g, _):
            pltpu.async_copy(eabuf.at[pl.ds(g * 128, 128)],
                             deg_acc.at[dstbuf.at[g]], sem, add=True)
            return 0
        lax.fori_loop(0, 80, g_fire, 0)

        def g_drain(g, _):
            pltpu.make_async_copy(eabuf.at[pl.ds(0, 128)],
                                  deg_acc.at[dstbuf.at[0]], sem).wait()
            return 0
        lax.fori_loop(0, 80, g_drain, 0)
        return 0
    lax.fori_loop(0, 5, sg_body, 0)
    plsc.subcore_barrier()

    nbase = pl.multiple_of(wid * NPT32, 128)
    pltpu.sync_copy(deg_acc.at[pl.ds(nbase, NPT32)], degbuf)
    pltpu.sync_copy(x_hbm.at[pl.ds(nbase, NPT32)], xbuf)

    def dis_body(i, _):
        disbuf[pl.ds(i * 16, 16)] = _rsqrt16(degbuf[pl.ds(i * 16, 16)])
        return 0
    lax.fori_loop(0, NPT32 // 16, dis_body, 0)
    pltpu.sync_copy(disbuf, dis_hbm.at[pl.ds(nbase, NPT32)])

    def j_body(j, _):
        pltpu.async_copy(emb_hbm.at[xbuf.at[pl.ds(j * 128, 128)]],
                         rows, sem).wait()

        def e_body(e16, _):
            w16 = disbuf[pl.ds(j * 128 + e16 * 16, 16)]
            for u in range(16):
                e = e16 * 16 + u
                w = w16[u]
                lobuf[e, pl.ds(0, 16)] = rows[e, pl.ds(0, 16)] * w
                lobuf[e, pl.ds(16, 16)] = rows[e, pl.ds(16, 16)] * w
                hibuf[e, pl.ds(0, 16)] = rows[e, pl.ds(32, 16)] * w
                hibuf[e, pl.ds(16, 16)] = rows[e, pl.ds(48, 16)] * w
            return 0
        lax.fori_loop(0, 8, e_body, 0)
        pltpu.sync_copy(lobuf, embcat_hbm.at[pl.ds(nbase + j * 128, 128)])
        pltpu.sync_copy(hibuf, embcat_hbm.at[pl.ds(N_PAD + nbase + j * 128, 128)])
        return 0
    lax.fori_loop(0, 13, j_body, 0)


@functools.partial(
    pl.kernel,
    out_type=jax.ShapeDtypeStruct((2 * N_PAD, 32), jnp.float32),
    mesh=_MESH,
    compiler_params=pltpu.CompilerParams(use_tc_tiling_on_sc=False),
    scratch_types=[
        pltpu.VMEM_SHARED((N_PAD, 32), jnp.float32),
        pltpu.VMEM((2048,), jnp.int32),
        pltpu.VMEM((16, 128), jnp.int32),
        pltpu.VMEM((2048,), jnp.float32),
        pltpu.VMEM((128, 32), jnp.float32),
        pltpu.VMEM((128, 32), jnp.float32),
        pltpu.SemaphoreType.DMA,
        pltpu.SemaphoreType.DMA,
        pltpu.SemaphoreType.DMA,
        pltpu.SemaphoreType.DMA,
    ],
)
def _sc_agg(embcat_hbm, src2f_hbm, dst2d_hbm, ea_hbm, znd_hbm,
            aggcat_hbm,
            acc, srcbuf, dstbuf, eabuf, rows0, rows1, sg0, sg1, ss0, ss1):
    c = lax.axis_index("c")
    s = lax.axis_index("s")

    pltpu.sync_copy(znd_hbm.at[pl.ds(s * NPT16, NPT16)],
                    acc.at[pl.ds(s * NPT16, NPT16)])
    plsc.subcore_barrier()

    def sg_body(sg, _):
        base = pl.multiple_of(s * EPT16 + sg * 2048, 1024)
        pltpu.sync_copy(src2f_hbm.at[pl.ds(c * E_PAD + base, 2048)], srcbuf)
        pltpu.sync_copy(ea_hbm.at[pl.ds(base, 2048)], eabuf)
        pltpu.sync_copy(dst2d_hbm.at[pl.ds(pl.multiple_of(base // 128, 8), 16)], dstbuf)

        def fire(g, buf, sem):
            pltpu.async_copy(embcat_hbm.at[srcbuf.at[pl.ds(g * 128, 128)]],
                             buf, sem)

        def wait_gather(buf, sem):
            pltpu.make_async_copy(
                embcat_hbm.at[srcbuf.at[pl.ds(0, 128)]], buf, sem).wait()

        def wait_scat(buf, sem):
            pltpu.make_async_copy(buf, acc.at[dstbuf.at[0]], sem).wait()

        def scale(g, buf):
            def e_body(e16, _):
                w16 = eabuf[pl.ds(g * 128 + e16 * 32, 16)]
                w16b = eabuf[pl.ds(g * 128 + e16 * 32 + 16, 16)]
                for u in range(16):
                    e16e = e16 * 32 + u
                    w = w16[u]
                    buf[e16e, pl.ds(0, 16)] = buf[e16e, pl.ds(0, 16)] * w
                    buf[e16e, pl.ds(16, 16)] = buf[e16e, pl.ds(16, 16)] * w
                for u in range(16):
                    e16e = e16 * 32 + 16 + u
                    w = w16b[u]
                    buf[e16e, pl.ds(0, 16)] = buf[e16e, pl.ds(0, 16)] * w
                    buf[e16e, pl.ds(16, 16)] = buf[e16e, pl.ds(16, 16)] * w
                return 0
            lax.fori_loop(0, 4, e_body, 0)

        fire(0, rows0, sg0)

        def pair_body(k, _):
            @pl.when(k > 0)
            def _():
                wait_scat(rows1, ss1)
            fire(2 * k + 1, rows1, sg1)
            wait_gather(rows0, sg0)
            scale(2 * k, rows0)
            pltpu.async_copy(rows0, acc.at[dstbuf.at[2 * k]], ss0, add=True)
            wait_gather(rows1, sg1)
            scale(2 * k + 1, rows1)
            pltpu.async_copy(rows1, acc.at[dstbuf.at[2 * k + 1]], ss1,
                             add=True)
            wait_scat(rows0, ss0)

            @pl.when(k < 7)
            def _():
                fire(2 * k + 2, rows0, sg0)
            return 0
        lax.fori_loop(0, 8, pair_body, 0)
        wait_scat(rows1, ss1)
        return 0
    lax.fori_loop(0, 25, sg_body, 0)
    plsc.subcore_barrier()

    pltpu.sync_copy(acc.at[pl.ds(s * NPT16, NPT16)],
                    aggcat_hbm.at[pl.ds(c * N_PAD + s * NPT16, NPT16)])


E_HALF = E_PAD // 2
EPT32H = E_HALF // TILES


def _make_sc_decode():
    return functools.partial(
        pl.kernel,
        out_type=jax.ShapeDtypeStruct((E_HALF, 64), jnp.bfloat16),
        mesh=_MESH,
        compiler_params=pltpu.CompilerParams(use_tc_tiling_on_sc=False),
        scratch_types=[
            pltpu.VMEM((2560,), jnp.int32),
            pltpu.VMEM((2560,), jnp.int32),
            pltpu.VMEM((256, 64), jnp.bfloat16),
            pltpu.VMEM((256, 64), jnp.bfloat16),
            pltpu.VMEM((256, 64), jnp.bfloat16),
            pltpu.VMEM((256, 64), jnp.bfloat16),
            pltpu.SemaphoreType.DMA,
            pltpu.SemaphoreType.DMA,
            pltpu.SemaphoreType.DMA,
            pltpu.SemaphoreType.DMA,
        ],
    )


@_make_sc_decode()
def _sc_decode_h(ab_hbm, srca_hbm, dstb_hbm,
               u_hbm,
               iabig, ibbig, a0, b0, a1, b1, sg0, sg1, sw0, sw1):
    c = lax.axis_index("c")
    s = lax.axis_index("s")
    wid = s * NC + c
    ebase = pl.multiple_of(wid * EPT32H, 512)

    def fire_g(goff, abuf, bbuf, sem):
        for j in range(2):
            pltpu.async_copy(
                ab_hbm.at[iabig.at[pl.ds(goff + j * 128, 128)]],
                abuf.at[pl.ds(j * 128, 128)], sem)
            pltpu.async_copy(
                ab_hbm.at[ibbig.at[pl.ds(goff + j * 128, 128)]],
                bbuf.at[pl.ds(j * 128, 128)], sem)

    def wait_g(abuf, bbuf, sem):
        for j in range(2):
            pltpu.make_async_copy(
                ab_hbm.at[iabig.at[pl.ds(0, 128)]],
                abuf.at[pl.ds(j * 128, 128)], sem).wait()
            pltpu.make_async_copy(
                ab_hbm.at[ibbig.at[pl.ds(0, 128)]],
                bbuf.at[pl.ds(j * 128, 128)], sem).wait()

    def relu_add(abuf, bbuf):
        def r_body(r4, _):
            for rr in range(4):
                r = r4 * 4 + rr
                for q in range(2):
                    va = abuf[r, pl.ds(q * 32, 32)]
                    vb = bbuf[r, pl.ds(q * 32, 32)]
                    abuf[r, pl.ds(q * 32, 32)] = jnp.maximum(va + vb, 0.0)
            return 0
        lax.fori_loop(0, 64, r_body, 0)

    def wait_w(abuf, sem):
        pltpu.make_async_copy(abuf, u_hbm.at[pl.ds(ebase, 256)], sem).wait()

    def m_body(m, _):
        moff = pl.multiple_of(ebase + m * 2560, 256)
        pltpu.sync_copy(srca_hbm.at[pl.ds(moff, 2560)], iabig)
        pltpu.sync_copy(dstb_hbm.at[pl.ds(moff, 2560)], ibbig)
        fire_g(0, a0, b0, sg0)

        def t_body(t, _):
            @pl.when(t > 0)
            def _():
                wait_w(a1, sw1)
            fire_g(t * 512 + 256, a1, b1, sg1)
            wait_g(a0, b0, sg0)
            relu_add(a0, b0)
            pltpu.async_copy(a0, u_hbm.at[pl.ds(moff + t * 512, 256)], sw0)
            wait_g(a1, b1, sg1)
            relu_add(a1, b1)
            pltpu.async_copy(
                a1, u_hbm.at[pl.ds(moff + t * 512 + 256, 256)], sw1)
            wait_w(a0, sw0)

            @pl.when(t < 4)
            def _():
                fire_g(t * 512 + 512, a0, b0, sg0)
            return 0
        lax.fori_loop(0, 5, t_body, 0)
        wait_w(a1, sw1)
        return 0
    lax.fori_loop(0, 5, m_body, 0)


def _elu(v):
    return jnp.where(v > 0, v, jnp.exp(jnp.minimum(v, 0.0)) - 1.0)


def _tc_dense_body(agglo_ref, agghi_ref, dis_ref, win_ref, bin_ref,
                   wlin_ref, blin_ref, w1_ref, b1_ref, a_ref, b_ref):
    dis = dis_ref[...]
    agg = jnp.concatenate([agglo_ref[...], agghi_ref[...]], axis=1) * dis
    t = _elu(jnp.dot(agg, win_ref[...],
                     preferred_element_type=jnp.float32) + bin_ref[...])
    nodes = _elu(jnp.dot(t, wlin_ref[...],
                         preferred_element_type=jnp.float32) + blin_ref[...])
    w1 = w1_ref[...]
    a_ref[...] = (jnp.dot(nodes, w1[0:64, :],
                          preferred_element_type=jnp.float32)
                  + b1_ref[...]).astype(jnp.bfloat16)
    b_ref[...] = jnp.dot(nodes, w1[64:128, :],
                         preferred_element_type=jnp.float32).astype(
                             jnp.bfloat16)


_BN = 2048


def _tc_dense(agg_cat, dis2, w_in, b_in2, w_lin, b_lin2, w1, b12):
    grid = (N_PAD // _BN,)
    full = lambda shape: pl.BlockSpec(shape, lambda i: (0, 0))
    return pl.pallas_call(
        _tc_dense_body,
        grid=grid,
        in_specs=[
            pl.BlockSpec((_BN, 32), lambda i: (i, 0)),
            pl.BlockSpec((_BN, 32), lambda i: (i + N_PAD // _BN, 0)),
            pl.BlockSpec((_BN, 1), lambda i: (i, 0)),
            full((EMB, HID)),
            full((1, HID)),
            full((HID, EMB)),
            full((1, EMB)),
            full((2 * EMB, EMB)),
            full((1, EMB)),
        ],
        out_specs=(
            pl.BlockSpec((_BN, 64), lambda i: (i, 0)),
            pl.BlockSpec((_BN, 64), lambda i: (i, 0)),
        ),
        out_shape=(
            jax.ShapeDtypeStruct((N_PAD, 64), jnp.bfloat16),
            jax.ShapeDtypeStruct((N_PAD, 64), jnp.bfloat16),
        ),
    )(agg_cat, agg_cat, dis2, w_in, b_in2, w_lin, b_lin2, w1, b12)


_BE = 8192


def _tc_mlp_body(u_ref, w2_ref, b2_ref, w3_ref, b3_ref, out_ref):
    h = jnp.maximum(jnp.dot(u_ref[...].astype(jnp.float32), w2_ref[...],
                            preferred_element_type=jnp.float32) + b2_ref[...],
                    0.0)
    res = jnp.dot(h, w3_ref[...].T, preferred_element_type=jnp.float32)
    out_ref[...] = res[:, 0] + b3_ref[0, 0]


def _tc_mlp(u, w2, b22, w3r, b32):
    grid = (E_HALF // _BE,)
    full = lambda shape: pl.BlockSpec(shape, lambda i: (0, 0))
    return pl.pallas_call(
        _tc_mlp_body,
        grid=grid,
        in_specs=[
            pl.BlockSpec((_BE, 64), lambda i: (i, 0)),
            full((EMB, EMB)),
            full((1, EMB)),
            full((1, EMB)),
            full((1, 1)),
        ],
        out_specs=pl.BlockSpec((_BE,), lambda i: (i,)),
        out_shape=jax.ShapeDtypeStruct((E_HALF,), jnp.float32),
    )(u, w2, b22, w3r, b32)


def kernel(x, edge_index, edge_attr, emb_table, W_in, b_in, W_lin, b_lin,
           W1, b1, W2, b2, W3, b3):
    src = edge_index[0]
    dst = edge_index[1]

    epad = E_PAD - E
    src_p = jnp.concatenate([src, jnp.zeros((epad,), jnp.int32)])
    dst_p = jnp.concatenate([dst, jnp.zeros((epad,), jnp.int32)])
    ea_p = jnp.concatenate([edge_attr, jnp.zeros((epad,), jnp.float32)])
    x_p = jnp.concatenate([x, jnp.zeros((N_PAD - N,), jnp.int32)])

    dst2d = dst_p.reshape(E_PAD // 128, 128)
    src2f = jnp.concatenate([src_p, src_p + N_PAD])
    dstb = dst_p + N_PAD

    zn = jnp.zeros((N_PAD,), jnp.float32)
    znd = jnp.zeros((N_PAD, 32), jnp.float32)

    dis, emb_cat = _sc_prep(emb_table, x_p, dst2d, ea_p, zn)
    agg_cat = _sc_agg(emb_cat, src2f, dst2d, ea_p, znd)

    a_mat, b_mat = _tc_dense(
        agg_cat, dis.reshape(N_PAD, 1),
        W_in, b_in.reshape(1, HID),
        W_lin, b_lin.reshape(1, EMB),
        W1, b1.reshape(1, EMB))

    ab = jnp.concatenate([a_mat, b_mat], axis=0)
    b22 = b2.reshape(1, EMB)
    w3r = W3.reshape(1, EMB)
    b32 = b3.reshape(1, 1)
    outs = []
    for h in range(2):
        u_h = _sc_decode_h(ab, src_p[h * E_HALF:(h + 1) * E_HALF],
                           dstb[h * E_HALF:(h + 1) * E_HALF])
        outs.append(_tc_mlp(u_h, W2, b22, w3r, b32))
    out = jnp.concatenate(outs)
    return out[:E]


# speedup vs baseline: 9.1833x; 1.0001x over previous
import functools

import jax
import jax.numpy as jnp
from jax import lax
from jax.experimental import pallas as pl
from jax.experimental.pallas import tpu as pltpu
from jax.experimental.pallas import tpu_sc as plsc

N = 50000
E = 800000
EMB = 64
HID = 128

NC = 2
NS = 16
TILES = NC * NS

N_PAD = 53248
E_PAD = 819200

NPT32 = N_PAD // TILES
NPT16 = N_PAD // NS
EPT16 = E_PAD // NS
EPT32 = E_PAD // TILES

_MESH = plsc.VectorSubcoreMesh(core_axis_name="c", subcore_axis_name="s")


def _rsqrt16(v):
    i = lax.bitcast_convert_type(v, jnp.int32)
    i = jnp.int32(0x5F3759DF) - (i >> 1)
    y = lax.bitcast_convert_type(i, jnp.float32)
    for _ in range(3):
        y = y * (jnp.float32(1.5) - jnp.float32(0.5) * v * y * y)
    return jnp.where(v > jnp.float32(0.0), y, jnp.float32(0.0))


@functools.partial(
    pl.kernel,
    out_type=(
        jax.ShapeDtypeStruct((N_PAD,), jnp.float32),
        jax.ShapeDtypeStruct((2 * N_PAD, 32), jnp.float32),
    ),
    mesh=_MESH,
    compiler_params=pltpu.CompilerParams(use_tc_tiling_on_sc=False),
    scratch_types=[
        pltpu.VMEM_SHARED((N_PAD,), jnp.float32),
        pltpu.VMEM((10240,), jnp.float32),
        pltpu.VMEM((80, 128), jnp.int32),
        pltpu.VMEM((NPT32,), jnp.float32),
        pltpu.VMEM((NPT32,), jnp.float32),
        pltpu.VMEM((NPT32,), jnp.int32),
        pltpu.VMEM((128, 64), jnp.float32),
        pltpu.VMEM((128, 32), jnp.float32),
        pltpu.VMEM((128, 32), jnp.float32),
        pltpu.SemaphoreType.DMA,
    ],
)
def _sc_prep(emb_hbm, x_hbm, dst2d_hbm, ea_hbm, zn_hbm,
             dis_hbm, embcat_hbm,
             deg_acc, eabuf, dstbuf, degbuf, disbuf, xbuf, rows, lobuf, hibuf,
             sem):
    c = lax.axis_index("c")
    s = lax.axis_index("s")
    wid = s * NC + c

    pltpu.sync_copy(zn_hbm.at[pl.ds(s * NPT16, NPT16)],
                    deg_acc.at[pl.ds(s * NPT16, NPT16)])
    plsc.subcore_barrier()

    def sg_body(sg, _):
        base = pl.multiple_of(s * EPT16 + sg * 10240, 1024)
        pltpu.sync_copy(ea_hbm.at[pl.ds(base, 10240)], eabuf)
        pltpu.sync_copy(dst2d_hbm.at[pl.ds(pl.multiple_of(base // 128, 8), 80)], dstbuf)

        def g_fire(g, _):
            pltpu.async_copy(eabuf.at[pl.ds(g * 128, 128)],
                             deg_acc.at[dstbuf.at[g]], sem, add=True)
            return 0
        lax.fori_loop(0, 80, g_fire, 0)

        def g_drain(g, _):
            pltpu.make_async_copy(eabuf.at[pl.ds(0, 128)],
                                  deg_acc.at[dstbuf.at[0]], sem).wait()
            return 0
        lax.fori_loop(0, 80, g_drain, 0)
        return 0
    lax.fori_loop(0, 5, sg_body, 0)
    plsc.subcore_barrier()

    nbase = pl.multiple_of(wid * NPT32, 128)
    pltpu.sync_copy(deg_acc.at[pl.ds(nbase, NPT32)], degbuf)
    pltpu.sync_copy(x_hbm.at[pl.ds(nbase, NPT32)], xbuf)

    def dis_body(i, _):
        disbuf[pl.ds(i * 16, 16)] = _rsqrt16(degbuf[pl.ds(i * 16, 16)])
        return 0
    lax.fori_loop(0, NPT32 // 16, dis_body, 0)
    pltpu.sync_copy(disbuf, dis_hbm.at[pl.ds(nbase, NPT32)])

    def j_body(j, _):
        pltpu.async_copy(emb_hbm.at[xbuf.at[pl.ds(j * 128, 128)]],
                         rows, sem).wait()

        def e_body(e16, _):
            for uu in range(2):
                w16 = disbuf[pl.ds(j * 128 + e16 * 32 + uu * 16, 16)]
                for u in range(16):
                    e = e16 * 32 + uu * 16 + u
                    w = w16[u]
                    lobuf[e, pl.ds(0, 16)] = rows[e, pl.ds(0, 16)] * w
                    lobuf[e, pl.ds(16, 16)] = rows[e, pl.ds(16, 16)] * w
                    hibuf[e, pl.ds(0, 16)] = rows[e, pl.ds(32, 16)] * w
                    hibuf[e, pl.ds(16, 16)] = rows[e, pl.ds(48, 16)] * w
            return 0
        lax.fori_loop(0, 4, e_body, 0)
        pltpu.sync_copy(lobuf, embcat_hbm.at[pl.ds(nbase + j * 128, 128)])
        pltpu.sync_copy(hibuf, embcat_hbm.at[pl.ds(N_PAD + nbase + j * 128, 128)])
        return 0
    lax.fori_loop(0, 13, j_body, 0)


@functools.partial(
    pl.kernel,
    out_type=jax.ShapeDtypeStruct((2 * N_PAD, 32), jnp.float32),
    mesh=_MESH,
    compiler_params=pltpu.CompilerParams(use_tc_tiling_on_sc=False),
    scratch_types=[
        pltpu.VMEM_SHARED((N_PAD, 32), jnp.float32),
        pltpu.VMEM((2048,), jnp.int32),
        pltpu.VMEM((16, 128), jnp.int32),
        pltpu.VMEM((2048,), jnp.float32),
        pltpu.VMEM((128, 32), jnp.float32),
        pltpu.VMEM((128, 32), jnp.float32),
        pltpu.SemaphoreType.DMA,
        pltpu.SemaphoreType.DMA,
        pltpu.SemaphoreType.DMA,
        pltpu.SemaphoreType.DMA,
    ],
)
def _sc_agg(embcat_hbm, src2f_hbm, dst2d_hbm, ea_hbm, znd_hbm,
            aggcat_hbm,
            acc, srcbuf, dstbuf, eabuf, rows0, rows1, sg0, sg1, ss0, ss1):
    c = lax.axis_index("c")
    s = lax.axis_index("s")

    pltpu.sync_copy(znd_hbm.at[pl.ds(s * NPT16, NPT16)],
                    acc.at[pl.ds(s * NPT16, NPT16)])
    plsc.subcore_barrier()

    def sg_body(sg, _):
        base = pl.multiple_of(s * EPT16 + sg * 2048, 1024)
        pltpu.sync_copy(src2f_hbm.at[pl.ds(c * E_PAD + base, 2048)], srcbuf)
        pltpu.sync_copy(ea_hbm.at[pl.ds(base, 2048)], eabuf)
        pltpu.sync_copy(dst2d_hbm.at[pl.ds(pl.multiple_of(base // 128, 8), 16)], dstbuf)

        def fire(g, buf, sem):
            pltpu.async_copy(embcat_hbm.at[srcbuf.at[pl.ds(g * 128, 128)]],
                             buf, sem)

        def wait_gather(buf, sem):
            pltpu.make_async_copy(
                embcat_hbm.at[srcbuf.at[pl.ds(0, 128)]], buf, sem).wait()

        def wait_scat(buf, sem):
            pltpu.make_async_copy(buf, acc.at[dstbuf.at[0]], sem).wait()

        def scale(g, buf):
            for blk in range(8):
                w16 = eabuf[pl.ds(g * 128 + blk * 16, 16)]
                for u in range(16):
                    e = blk * 16 + u
                    w = w16[u]
                    buf[e, pl.ds(0, 16)] = buf[e, pl.ds(0, 16)] * w
                    buf[e, pl.ds(16, 16)] = buf[e, pl.ds(16, 16)] * w

        fire(0, rows0, sg0)

        def pair_body(k, _):
            @pl.when(k > 0)
            def _():
                wait_scat(rows1, ss1)
            fire(2 * k + 1, rows1, sg1)
            wait_gather(rows0, sg0)
            scale(2 * k, rows0)
            pltpu.async_copy(rows0, acc.at[dstbuf.at[2 * k]], ss0, add=True)
            wait_gather(rows1, sg1)
            scale(2 * k + 1, rows1)
            pltpu.async_copy(rows1, acc.at[dstbuf.at[2 * k + 1]], ss1,
                             add=True)
            wait_scat(rows0, ss0)

            @pl.when(k < 7)
            def _():
                fire(2 * k + 2, rows0, sg0)
            return 0
        lax.fori_loop(0, 8, pair_body, 0)
        wait_scat(rows1, ss1)
        return 0
    lax.fori_loop(0, 25, sg_body, 0)
    plsc.subcore_barrier()

    pltpu.sync_copy(acc.at[pl.ds(s * NPT16, NPT16)],
                    aggcat_hbm.at[pl.ds(c * N_PAD + s * NPT16, NPT16)])


E_HALF = E_PAD // 2
EPT32H = E_HALF // TILES


def _make_sc_decode():
    return functools.partial(
        pl.kernel,
        out_type=jax.ShapeDtypeStruct((E_HALF, 64), jnp.bfloat16),
        mesh=_MESH,
        compiler_params=pltpu.CompilerParams(use_tc_tiling_on_sc=False),
        scratch_types=[
            pltpu.VMEM((2560,), jnp.int32),
            pltpu.VMEM((2560,), jnp.int32),
            pltpu.VMEM((256, 64), jnp.bfloat16),
            pltpu.VMEM((256, 64), jnp.bfloat16),
            pltpu.VMEM((256, 64), jnp.bfloat16),
            pltpu.VMEM((256, 64), jnp.bfloat16),
            pltpu.SemaphoreType.DMA,
            pltpu.SemaphoreType.DMA,
            pltpu.SemaphoreType.DMA,
            pltpu.SemaphoreType.DMA,
        ],
    )


@_make_sc_decode()
def _sc_decode_h(ab_hbm, srca_hbm, dstb_hbm,
               u_hbm,
               iabig, ibbig, a0, b0, a1, b1, sg0, sg1, sw0, sw1):
    c = lax.axis_index("c")
    s = lax.axis_index("s")
    wid = s * NC + c
    ebase = pl.multiple_of(wid * EPT32H, 512)

    def fire_g(goff, abuf, bbuf, sem):
        for j in range(2):
            pltpu.async_copy(
                ab_hbm.at[iabig.at[pl.ds(goff + j * 128, 128)]],
                abuf.at[pl.ds(j * 128, 128)], sem)
            pltpu.async_copy(
                ab_hbm.at[ibbig.at[pl.ds(goff + j * 128, 128)]],
                bbuf.at[pl.ds(j * 128, 128)], sem)

    def wait_g(abuf, bbuf, sem):
        for j in range(2):
            pltpu.make_async_copy(
                ab_hbm.at[iabig.at[pl.ds(0, 128)]],
                abuf.at[pl.ds(j * 128, 128)], sem).wait()
            pltpu.make_async_copy(
                ab_hbm.at[ibbig.at[pl.ds(0, 128)]],
                bbuf.at[pl.ds(j * 128, 128)], sem).wait()

    def relu_add(abuf, bbuf):
        def r_body(r4, _):
            for rr in range(4):
                r = r4 * 4 + rr
                for q in range(2):
                    va = abuf[r, pl.ds(q * 32, 32)]
                    vb = bbuf[r, pl.ds(q * 32, 32)]
                    abuf[r, pl.ds(q * 32, 32)] = jnp.maximum(va + vb, 0.0)
            return 0
        lax.fori_loop(0, 64, r_body, 0)

    def wait_w(abuf, sem):
        pltpu.make_async_copy(abuf, u_hbm.at[pl.ds(ebase, 256)], sem).wait()

    def m_body(m, _):
        moff = pl.multiple_of(ebase + m * 2560, 256)
        pltpu.sync_copy(srca_hbm.at[pl.ds(moff, 2560)], iabig)
        pltpu.sync_copy(dstb_hbm.at[pl.ds(moff, 2560)], ibbig)
        fire_g(0, a0, b0, sg0)

        def t_body(t, _):
            @pl.when(t > 0)
            def _():
                wait_w(a1, sw1)
            fire_g(t * 512 + 256, a1, b1, sg1)
            wait_g(a0, b0, sg0)
            relu_add(a0, b0)
            pltpu.async_copy(a0, u_hbm.at[pl.ds(moff + t * 512, 256)], sw0)
            wait_g(a1, b1, sg1)
            relu_add(a1, b1)
            pltpu.async_copy(
                a1, u_hbm.at[pl.ds(moff + t * 512 + 256, 256)], sw1)
            wait_w(a0, sw0)

            @pl.when(t < 4)
            def _():
                fire_g(t * 512 + 512, a0, b0, sg0)
            return 0
        lax.fori_loop(0, 5, t_body, 0)
        wait_w(a1, sw1)
        return 0
    lax.fori_loop(0, 5, m_body, 0)


def _elu(v):
    return jnp.where(v > 0, v, jnp.exp(jnp.minimum(v, 0.0)) - 1.0)


def _tc_dense_body(agglo_ref, agghi_ref, dis_ref, win_ref, bin_ref,
                   wlin_ref, blin_ref, w1_ref, b1_ref, a_ref, b_ref):
    dis = dis_ref[...]
    agg = jnp.concatenate([agglo_ref[...], agghi_ref[...]], axis=1) * dis
    t = _elu(jnp.dot(agg, win_ref[...],
                     preferred_element_type=jnp.float32) + bin_ref[...])
    nodes = _elu(jnp.dot(t, wlin_ref[...],
                         preferred_element_type=jnp.float32) + blin_ref[...])
    w1 = w1_ref[...]
    a_ref[...] = (jnp.dot(nodes, w1[0:64, :],
                          preferred_element_type=jnp.float32)
                  + b1_ref[...]).astype(jnp.bfloat16)
    b_ref[...] = jnp.dot(nodes, w1[64:128, :],
                         preferred_element_type=jnp.float32).astype(
                             jnp.bfloat16)


_BN = 2048


def _tc_dense(agg_cat, dis2, w_in, b_in2, w_lin, b_lin2, w1, b12):
    grid = (N_PAD // _BN,)
    full = lambda shape: pl.BlockSpec(shape, lambda i: (0, 0))
    return pl.pallas_call(
        _tc_dense_body,
        grid=grid,
        in_specs=[
            pl.BlockSpec((_BN, 32), lambda i: (i, 0)),
            pl.BlockSpec((_BN, 32), lambda i: (i + N_PAD // _BN, 0)),
            pl.BlockSpec((_BN, 1), lambda i: (i, 0)),
            full((EMB, HID)),
            full((1, HID)),
            full((HID, EMB)),
            full((1, EMB)),
            full((2 * EMB, EMB)),
            full((1, EMB)),
        ],
        out_specs=(
            pl.BlockSpec((_BN, 64), lambda i: (i, 0)),
            pl.BlockSpec((_BN, 64), lambda i: (i, 0)),
        ),
        out_shape=(
            jax.ShapeDtypeStruct((N_PAD, 64), jnp.bfloat16),
            jax.ShapeDtypeStruct((N_PAD, 64), jnp.bfloat16),
        ),
    )(agg_cat, agg_cat, dis2, w_in, b_in2, w_lin, b_lin2, w1, b12)


_BE = 8192


def _tc_mlp_body(u_ref, w2_ref, b2_ref, w3_ref, b3_ref, out_ref):
    h = jnp.maximum(jnp.dot(u_ref[...].astype(jnp.float32), w2_ref[...],
                            preferred_element_type=jnp.float32) + b2_ref[...],
                    0.0)
    res = jnp.dot(h, w3_ref[...].T, preferred_element_type=jnp.float32)
    out_ref[...] = res[:, 0] + b3_ref[0, 0]


def _tc_mlp(u, w2, b22, w3r, b32):
    grid = (E_HALF // _BE,)
    full = lambda shape: pl.BlockSpec(shape, lambda i: (0, 0))
    return pl.pallas_call(
        _tc_mlp_body,
        grid=grid,
        in_specs=[
            pl.BlockSpec((_BE, 64), lambda i: (i, 0)),
            full((EMB, EMB)),
            full((1, EMB)),
            full((1, EMB)),
            full((1, 1)),
        ],
        out_specs=pl.BlockSpec((_BE,), lambda i: (i,)),
        out_shape=jax.ShapeDtypeStruct((E_HALF,), jnp.float32),
    )(u, w2, b22, w3r, b32)


def kernel(x, edge_index, edge_attr, emb_table, W_in, b_in, W_lin, b_lin,
           W1, b1, W2, b2, W3, b3):
    src = edge_index[0]
    dst = edge_index[1]

    epad = E_PAD - E
    src_p = jnp.concatenate([src, jnp.zeros((epad,), jnp.int32)])
    dst_p = jnp.concatenate([dst, jnp.zeros((epad,), jnp.int32)])
    ea_p = jnp.concatenate([edge_attr, jnp.zeros((epad,), jnp.float32)])
    x_p = jnp.concatenate([x, jnp.zeros((N_PAD - N,), jnp.int32)])

    dst2d = dst_p.reshape(E_PAD // 128, 128)
    src2f = jnp.concatenate([src_p, src_p + N_PAD])
    dstb = dst_p + N_PAD

    zn = jnp.zeros((N_PAD,), jnp.float32)
    znd = jnp.zeros((N_PAD, 32), jnp.float32)

    dis, emb_cat = _sc_prep(emb_table, x_p, dst2d, ea_p, zn)
    agg_cat = _sc_agg(emb_cat, src2f, dst2d, ea_p, znd)

    a_mat, b_mat = _tc_dense(
        agg_cat, dis.reshape(N_PAD, 1),
        W_in, b_in.reshape(1, HID),
        W_lin, b_lin.reshape(1, EMB),
        W1, b1.reshape(1, EMB))

    ab = jnp.concatenate([a_mat, b_mat], axis=0)
    b22 = b2.reshape(1, EMB)
    w3r = W3.reshape(1, EMB)
    b32 = b3.reshape(1, 1)
    outs = []
    for h in range(2):
        u_h = _sc_decode_h(ab, src_p[h * E_HALF:(h + 1) * E_HALF],
                           dstb[h * E_HALF:(h + 1) * E_HALF])
        outs.append(_tc_mlp(u_h, W2, b22, w3r, b32))
    out = jnp.concatenate(outs)
    return out[:E]


# speedup vs baseline: 9.2237x; 1.0044x over previous
import functools

import jax
import jax.numpy as jnp
from jax import lax
from jax.experimental import pallas as pl
from jax.experimental.pallas import tpu as pltpu
from jax.experimental.pallas import tpu_sc as plsc

N = 50000
E = 800000
EMB = 64
HID = 128

NC = 2
NS = 16
TILES = NC * NS

N_PAD = 53248
E_PAD = 819200

NPT32 = N_PAD // TILES
NPT16 = N_PAD // NS
EPT16 = E_PAD // NS
EPT32 = E_PAD // TILES

_MESH = plsc.VectorSubcoreMesh(core_axis_name="c", subcore_axis_name="s")


def _rsqrt16(v):
    i = lax.bitcast_convert_type(v, jnp.int32)
    i = jnp.int32(0x5F3759DF) - (i >> 1)
    y = lax.bitcast_convert_type(i, jnp.float32)
    for _ in range(3):
        y = y * (jnp.float32(1.5) - jnp.float32(0.5) * v * y * y)
    return jnp.where(v > jnp.float32(0.0), y, jnp.float32(0.0))


@functools.partial(
    pl.kernel,
    out_type=(
        jax.ShapeDtypeStruct((N_PAD,), jnp.float32),
        jax.ShapeDtypeStruct((2 * N_PAD, 32), jnp.float32),
    ),
    mesh=_MESH,
    compiler_params=pltpu.CompilerParams(use_tc_tiling_on_sc=False),
    scratch_types=[
        pltpu.VMEM_SHARED((N_PAD,), jnp.float32),
        pltpu.VMEM((10240,), jnp.float32),
        pltpu.VMEM((80, 128), jnp.int32),
        pltpu.VMEM((NPT32,), jnp.float32),
        pltpu.VMEM((NPT32,), jnp.float32),
        pltpu.VMEM((NPT32,), jnp.int32),
        pltpu.VMEM((128, 64), jnp.float32),
        pltpu.VMEM((128, 32), jnp.float32),
        pltpu.VMEM((128, 32), jnp.float32),
        pltpu.SemaphoreType.DMA,
    ],
)
def _sc_prep(emb_hbm, x_hbm, dst2d_hbm, ea_hbm, zn_hbm,
             dis_hbm, embcat_hbm,
             deg_acc, eabuf, dstbuf, degbuf, disbuf, xbuf, rows, lobuf, hibuf,
             sem):
    c = lax.axis_index("c")
    s = lax.axis_index("s")
    wid = s * NC + c

    pltpu.sync_copy(zn_hbm.at[pl.ds(s * NPT16, NPT16)],
                    deg_acc.at[pl.ds(s * NPT16, NPT16)])
    plsc.subcore_barrier()

    def sg_body(sg, _):
        base = pl.multiple_of(s * EPT16 + sg * 10240, 1024)
        pltpu.sync_copy(ea_hbm.at[pl.ds(base, 10240)], eabuf)
        pltpu.sync_copy(dst2d_hbm.at[pl.ds(pl.multiple_of(base // 128, 8), 80)], dstbuf)

        def g_fire(g, _):
            pltpu.async_copy(eabuf.at[pl.ds(g * 128, 128)],
                             deg_acc.at[dstbuf.at[g]], sem, add=True)
            return 0
        lax.fori_loop(0, 80, g_fire, 0)

        def g_drain(g, _):
            pltpu.make_async_copy(eabuf.at[pl.ds(0, 128)],
                                  deg_acc.at[dstbuf.at[0]], sem).wait()
            return 0
        lax.fori_loop(0, 80, g_drain, 0)
        return 0
    lax.fori_loop(0, 5, sg_body, 0)
    plsc.subcore_barrier()

    nbase = pl.multiple_of(wid * NPT32, 128)
    pltpu.sync_copy(deg_acc.at[pl.ds(nbase, NPT32)], degbuf)
    pltpu.sync_copy(x_hbm.at[pl.ds(nbase, NPT32)], xbuf)

    def dis_body(i, _):
        disbuf[pl.ds(i * 16, 16)] = _rsqrt16(degbuf[pl.ds(i * 16, 16)])
        return 0
    lax.fori_loop(0, NPT32 // 16, dis_body, 0)
    pltpu.sync_copy(disbuf, dis_hbm.at[pl.ds(nbase, NPT32)])

    def j_body(j, _):
        pltpu.async_copy(emb_hbm.at[xbuf.at[pl.ds(j * 128, 128)]],
                         rows, sem).wait()

        def e_body(e16, _):
            for uu in range(2):
                w16 = disbuf[pl.ds(j * 128 + e16 * 32 + uu * 16, 16)]
                for u in range(16):
                    e = e16 * 32 + uu * 16 + u
                    w = w16[u]
                    lobuf[e, pl.ds(0, 16)] = rows[e, pl.ds(0, 16)] * w
                    lobuf[e, pl.ds(16, 16)] = rows[e, pl.ds(16, 16)] * w
                    hibuf[e, pl.ds(0, 16)] = rows[e, pl.ds(32, 16)] * w
                    hibuf[e, pl.ds(16, 16)] = rows[e, pl.ds(48, 16)] * w
            return 0
        lax.fori_loop(0, 4, e_body, 0)
        pltpu.sync_copy(lobuf, embcat_hbm.at[pl.ds(nbase + j * 128, 128)])
        pltpu.sync_copy(hibuf, embcat_hbm.at[pl.ds(N_PAD + nbase + j * 128, 128)])
        return 0
    lax.fori_loop(0, 13, j_body, 0)


@functools.partial(
    pl.kernel,
    out_type=jax.ShapeDtypeStruct((2 * N_PAD, 32), jnp.float32),
    mesh=_MESH,
    compiler_params=pltpu.CompilerParams(use_tc_tiling_on_sc=False),
    scratch_types=[
        pltpu.VMEM_SHARED((N_PAD, 32), jnp.float32),
        pltpu.VMEM((2048,), jnp.int32),
        pltpu.VMEM((16, 128), jnp.int32),
        pltpu.VMEM((2048,), jnp.float32),
        pltpu.VMEM((128, 32), jnp.float32),
        pltpu.VMEM((128, 32), jnp.float32),
        pltpu.SemaphoreType.DMA,
        pltpu.SemaphoreType.DMA,
        pltpu.SemaphoreType.DMA,
        pltpu.SemaphoreType.DMA,
    ],
)
def _sc_agg(embcat_hbm, src2f_hbm, dst2d_hbm, ea_hbm, znd_hbm,
            aggcat_hbm,
            acc, srcbuf, dstbuf, eabuf, rows0, rows1, sg0, sg1, ss0, ss1):
    c = lax.axis_index("c")
    s = lax.axis_index("s")

    pltpu.sync_copy(znd_hbm.at[pl.ds(s * NPT16, NPT16)],
                    acc.at[pl.ds(s * NPT16, NPT16)])
    plsc.subcore_barrier()

    def sg_body(sg, _):
        base = pl.multiple_of(s * EPT16 + sg * 2048, 1024)
        pltpu.sync_copy(src2f_hbm.at[pl.ds(c * E_PAD + base, 2048)], srcbuf)
        pltpu.sync_copy(ea_hbm.at[pl.ds(base, 2048)], eabuf)
        pltpu.sync_copy(dst2d_hbm.at[pl.ds(pl.multiple_of(base // 128, 8), 16)], dstbuf)

        def fire(g, buf, sem):
            pltpu.async_copy(embcat_hbm.at[srcbuf.at[pl.ds(g * 128, 128)]],
                             buf, sem)

        def wait_gather(buf, sem):
            pltpu.make_async_copy(
                embcat_hbm.at[srcbuf.at[pl.ds(0, 128)]], buf, sem).wait()

        def wait_scat(buf, sem):
            pltpu.make_async_copy(buf, acc.at[dstbuf.at[0]], sem).wait()

        def scale(g, buf):
            for blk in range(8):
                w16 = eabuf[pl.ds(g * 128 + blk * 16, 16)]
                for u in range(16):
                    e = blk * 16 + u
                    w = w16[u]
                    buf[e, pl.ds(0, 16)] = buf[e, pl.ds(0, 16)] * w
                    buf[e, pl.ds(16, 16)] = buf[e, pl.ds(16, 16)] * w

        fire(0, rows0, sg0)

        def pair_body(k, _):
            @pl.when(k > 0)
            def _():
                wait_scat(rows1, ss1)
            fire(2 * k + 1, rows1, sg1)
            wait_gather(rows0, sg0)
            scale(2 * k, rows0)
            pltpu.async_copy(rows0, acc.at[dstbuf.at[2 * k]], ss0, add=True)
            wait_gather(rows1, sg1)
            scale(2 * k + 1, rows1)
            pltpu.async_copy(rows1, acc.at[dstbuf.at[2 * k + 1]], ss1,
                             add=True)
            wait_scat(rows0, ss0)

            @pl.when(k < 7)
            def _():
                fire(2 * k + 2, rows0, sg0)
            return 0
        lax.fori_loop(0, 8, pair_body, 0)
        wait_scat(rows1, ss1)
        return 0
    lax.fori_loop(0, 25, sg_body, 0)
    plsc.subcore_barrier()

    pltpu.sync_copy(acc.at[pl.ds(s * NPT16, NPT16)],
                    aggcat_hbm.at[pl.ds(c * N_PAD + s * NPT16, NPT16)])


E_HALF = E_PAD // 2
EPT32H = E_HALF // TILES


def _make_sc_decode():
    return functools.partial(
        pl.kernel,
        out_type=jax.ShapeDtypeStruct((E_HALF, 64), jnp.bfloat16),
        mesh=_MESH,
        compiler_params=pltpu.CompilerParams(use_tc_tiling_on_sc=False),
        scratch_types=[
            pltpu.VMEM((2560,), jnp.int32),
            pltpu.VMEM((2560,), jnp.int32),
            pltpu.VMEM((256, 64), jnp.bfloat16),
            pltpu.VMEM((256, 64), jnp.bfloat16),
            pltpu.VMEM((256, 64), jnp.bfloat16),
            pltpu.VMEM((256, 64), jnp.bfloat16),
            pltpu.SemaphoreType.DMA,
            pltpu.SemaphoreType.DMA,
            pltpu.SemaphoreType.DMA,
            pltpu.SemaphoreType.DMA,
        ],
    )


@_make_sc_decode()
def _sc_decode_h(ab_hbm, srca_hbm, dstb_hbm,
               u_hbm,
               iabig, ibbig, a0, b0, a1, b1, sg0, sg1, sw0, sw1):
    c = lax.axis_index("c")
    s = lax.axis_index("s")
    wid = s * NC + c
    ebase = pl.multiple_of(wid * EPT32H, 512)

    def fire_g(goff, abuf, bbuf, sem):
        for j in range(2):
            pltpu.async_copy(
                ab_hbm.at[iabig.at[pl.ds(goff + j * 128, 128)]],
                abuf.at[pl.ds(j * 128, 128)], sem)
            pltpu.async_copy(
                ab_hbm.at[ibbig.at[pl.ds(goff + j * 128, 128)]],
                bbuf.at[pl.ds(j * 128, 128)], sem)

    def wait_g(abuf, bbuf, sem):
        for j in range(2):
            pltpu.make_async_copy(
                ab_hbm.at[iabig.at[pl.ds(0, 128)]],
                abuf.at[pl.ds(j * 128, 128)], sem).wait()
            pltpu.make_async_copy(
                ab_hbm.at[ibbig.at[pl.ds(0, 128)]],
                bbuf.at[pl.ds(j * 128, 128)], sem).wait()

    def relu_add(abuf, bbuf):
        def r_body(r4, _):
            for rr in range(4):
                r = r4 * 4 + rr
                for q in range(2):
                    va = abuf[r, pl.ds(q * 32, 32)]
                    vb = bbuf[r, pl.ds(q * 32, 32)]
                    abuf[r, pl.ds(q * 32, 32)] = jnp.maximum(va + vb, 0.0)
            return 0
        lax.fori_loop(0, 64, r_body, 0)

    def wait_w(abuf, sem):
        pltpu.make_async_copy(abuf, u_hbm.at[pl.ds(ebase, 256)], sem).wait()

    def m_body(m, _):
        moff = pl.multiple_of(ebase + m * 2560, 256)
        pltpu.sync_copy(srca_hbm.at[pl.ds(moff, 2560)], iabig)
        pltpu.sync_copy(dstb_hbm.at[pl.ds(moff, 2560)], ibbig)
        fire_g(0, a0, b0, sg0)

        def t_body(t, _):
            @pl.when(t > 0)
            def _():
                wait_w(a1, sw1)
            fire_g(t * 512 + 256, a1, b1, sg1)
            wait_g(a0, b0, sg0)
            relu_add(a0, b0)
            pltpu.async_copy(a0, u_hbm.at[pl.ds(moff + t * 512, 256)], sw0)
            wait_g(a1, b1, sg1)
            relu_add(a1, b1)
            pltpu.async_copy(
                a1, u_hbm.at[pl.ds(moff + t * 512 + 256, 256)], sw1)
            wait_w(a0, sw0)

            @pl.when(t < 4)
            def _():
                fire_g(t * 512 + 512, a0, b0, sg0)
            return 0
        lax.fori_loop(0, 5, t_body, 0)
        wait_w(a1, sw1)
        return 0
    lax.fori_loop(0, 5, m_body, 0)


def _elu(v):
    return jnp.where(v > 0, v, jnp.exp(jnp.minimum(v, 0.0)) - 1.0)


def _tc_dense_body(agglo_ref, agghi_ref, dis_ref, win_ref, bin_ref,
                   wlin_ref, blin_ref, w1_ref, b1_ref, a_ref, b_ref):
    dis = dis_ref[...]
    agg = jnp.concatenate([agglo_ref[...], agghi_ref[...]], axis=1) * dis
    t = _elu(jnp.dot(agg, win_ref[...],
                     preferred_element_type=jnp.float32) + bin_ref[...])
    nodes = _elu(jnp.dot(t, wlin_ref[...],
                         preferred_element_type=jnp.float32) + blin_ref[...])
    w1 = w1_ref[...]
    a_ref[...] = (jnp.dot(nodes, w1[0:64, :],
                          preferred_element_type=jnp.float32)
                  + b1_ref[...]).astype(jnp.bfloat16)
    b_ref[...] = jnp.dot(nodes, w1[64:128, :],
                         preferred_element_type=jnp.float32).astype(
                             jnp.bfloat16)


_BN = 2048


def _tc_dense(agg_cat, dis2, w_in, b_in2, w_lin, b_lin2, w1, b12):
    grid = (N_PAD // _BN,)
    full = lambda shape: pl.BlockSpec(shape, lambda i: (0, 0))
    return pl.pallas_call(
        _tc_dense_body,
        grid=grid,
        in_specs=[
            pl.BlockSpec((_BN, 32), lambda i: (i, 0)),
            pl.BlockSpec((_BN, 32), lambda i: (i + N_PAD // _BN, 0)),
            pl.BlockSpec((_BN, 1), lambda i: (i, 0)),
            full((EMB, HID)),
            full((1, HID)),
            full((HID, EMB)),
            full((1, EMB)),
            full((2 * EMB, EMB)),
            full((1, EMB)),
        ],
        out_specs=(
            pl.BlockSpec((_BN, 64), lambda i: (i, 0)),
            pl.BlockSpec((_BN, 64), lambda i: (i, 0)),
        ),
        out_shape=(
            jax.ShapeDtypeStruct((N_PAD, 64), jnp.bfloat16),
            jax.ShapeDtypeStruct((N_PAD, 64), jnp.bfloat16),
        ),
    )(agg_cat, agg_cat, dis2, w_in, b_in2, w_lin, b_lin2, w1, b12)


_BE = 8192


def _tc_mlp_body(u_ref, w2_ref, b2_ref, w3_ref, b3_ref, out_ref):
    h = jnp.maximum(jnp.dot(u_ref[...], w2_ref[...].astype(jnp.bfloat16),
                            preferred_element_type=jnp.float32) + b2_ref[...],
                    0.0)
    res = jnp.dot(h, w3_ref[...].T, preferred_element_type=jnp.float32)
    out_ref[...] = res[:, 0] + b3_ref[0, 0]


def _tc_mlp(u, w2, b22, w3r, b32):
    grid = (E_HALF // _BE,)
    full = lambda shape: pl.BlockSpec(shape, lambda i: (0, 0))
    return pl.pallas_call(
        _tc_mlp_body,
        grid=grid,
        in_specs=[
            pl.BlockSpec((_BE, 64), lambda i: (i, 0)),
            full((EMB, EMB)),
            full((1, EMB)),
            full((1, EMB)),
            full((1, 1)),
        ],
        out_specs=pl.BlockSpec((_BE,), lambda i: (i,)),
        out_shape=jax.ShapeDtypeStruct((E_HALF,), jnp.float32),
    )(u, w2, b22, w3r, b32)


def kernel(x, edge_index, edge_attr, emb_table, W_in, b_in, W_lin, b_lin,
           W1, b1, W2, b2, W3, b3):
    src = edge_index[0]
    dst = edge_index[1]

    epad = E_PAD - E
    src_p = jnp.concatenate([src, jnp.zeros((epad,), jnp.int32)])
    dst_p = jnp.concatenate([dst, jnp.zeros((epad,), jnp.int32)])
    ea_p = jnp.concatenate([edge_attr, jnp.zeros((epad,), jnp.float32)])
    x_p = jnp.concatenate([x, jnp.zeros((N_PAD - N,), jnp.int32)])

    dst2d = dst_p.reshape(E_PAD // 128, 128)
    src2f = jnp.concatenate([src_p, src_p + N_PAD])
    dstb = dst_p + N_PAD

    zn = jnp.zeros((N_PAD,), jnp.float32)
    znd = jnp.zeros((N_PAD, 32), jnp.float32)

    dis, emb_cat = _sc_prep(emb_table, x_p, dst2d, ea_p, zn)
    agg_cat = _sc_agg(emb_cat, src2f, dst2d, ea_p, znd)

    a_mat, b_mat = _tc_dense(
        agg_cat, dis.reshape(N_PAD, 1),
        W_in, b_in.reshape(1, HID),
        W_lin, b_lin.reshape(1, EMB),
        W1, b1.reshape(1, EMB))

    ab = jnp.concatenate([a_mat, b_mat], axis=0)
    b22 = b2.reshape(1, EMB)
    w3r = W3.reshape(1, EMB)
    b32 = b3.reshape(1, 1)
    outs = []
    for h in range(2):
        u_h = _sc_decode_h(ab, src_p[h * E_HALF:(h + 1) * E_HALF],
                           dstb[h * E_HALF:(h + 1) * E_HALF])
        outs.append(_tc_mlp(u_h, W2, b22, w3r, b32))
    out = jnp.concatenate(outs)
    return out[:E]
